# Initial kernel scaffold; baseline (speedup 1.0000x reference)
import jax
import jax.numpy as jnp
from jax.experimental import pallas as pl


def kernel(x_g1, edge_index_g1, W1, b1, x_g2, edge_index_g2, edge_type_g2, bases, comp, root, b2):
    raise NotImplementedError("write your pallas kernel here")


# speedup vs baseline: 7.7530x; 7.7530x over previous
import jax
import jax.numpy as jnp
from jax import lax
from jax.experimental import pallas as pl
from jax.experimental.pallas import tpu as pltpu
from jax.experimental.pallas import tpu_sc as plsc

F32 = jnp.float32

N1 = 10000
E1 = 320000
D = 128
HD = D // 2
N2 = 256
E2 = 2048
NUM_REL = 2048
NB = 30
NBP = 32

NC = 2
NS = 16
NW = NC * NS

CH = 128
CPT1 = 80
EPT1 = CPT1 * CH
E1P = EPT1 * NW

RPT = 632
AGG_ROWS = NS * RPT

A_ROWS = N2 * N2
A_RPT = A_ROWS // NS
EPT2 = E2 // NS

_HIGH = lax.Precision.HIGHEST


def _sc_count_body(col3d, etype2d, dst2d, src2d, comp8, zeros16, zeros8, ones16,
                   deg_out, a_out, cnt_out,
                   degmat_sh, a_sh, cnt_sh,
                   colbuf, compbuf, onesbuf, zbuf, zbuf8, ebuf, dbuf, sbuf,
                   fbuf):
    cid = lax.axis_index("c")
    sid = lax.axis_index("s")
    wid = cid * NS + sid

    pltpu.sync_copy(zeros16, zbuf)
    pltpu.sync_copy(zeros8, zbuf8)
    pltpu.sync_copy(ones16, onesbuf)
    pltpu.sync_copy(zbuf, degmat_sh.at[pl.ds(sid * RPT, RPT)])

    @pl.when(jnp.logical_and(cid == 0, sid == 0))
    def _zero_cnt():
        pltpu.sync_copy(zbuf.at[pl.ds(0, N2)], cnt_sh)

    pltpu.sync_copy(col3d.at[wid], colbuf)
    pltpu.sync_copy(etype2d.at[sid], ebuf.at[0])
    pltpu.sync_copy(dst2d.at[sid], dbuf.at[0])
    pltpu.sync_copy(src2d.at[sid], sbuf.at[0])

    for k in range(EPT2 // 16):
        sl = pl.ds(k * 16, 16)
        fbuf[0, sl] = dbuf[0, sl] * N2 + sbuf[0, sl]

    plsc.subcore_barrier()

    def deg_step(j, carry):
        pltpu.sync_copy(onesbuf, degmat_sh.at[colbuf.at[j]], add=True)
        return carry

    lax.fori_loop(0, CPT1, deg_step, 0)

    @pl.when(cid == 0)
    def _cnt():
        pltpu.sync_copy(onesbuf, cnt_sh.at[dbuf.at[0]], add=True)

    for g in range(2):
        gidx = cid * 2 + g
        for k in range(EPT2 // 16):
            sl = pl.ds(k * 16, 16)
            ebuf[1, sl] = ebuf[0, sl] + gidx * NUM_REL
        pltpu.sync_copy(zbuf8, a_sh.at[pl.ds(sid * A_RPT, A_RPT)])
        plsc.subcore_barrier()
        pltpu.sync_copy(comp8.at[ebuf.at[1]], compbuf)
        pltpu.sync_copy(compbuf, a_sh.at[fbuf.at[0]], add=True)
        plsc.subcore_barrier()
        pltpu.sync_copy(a_sh.at[pl.ds(sid * A_RPT, A_RPT)],
                        a_out.at[gidx, pl.ds(sid * A_RPT, A_RPT)])

    plsc.subcore_barrier()
    pltpu.sync_copy(degmat_sh.at[pl.ds(sid * RPT, RPT)],
                    deg_out.at[cid, pl.ds(sid * RPT, RPT)])

    @pl.when(cid == 0)
    def _dump_cnt():
        pltpu.sync_copy(cnt_sh.at[pl.ds(sid * 16, 16)],
                        cnt_out.at[pl.ds(sid * 16, 16)])


def _sc_count(col3d, etype2d, dst2d, src2d, comp8, zeros16, zeros8, ones16):
    mesh = plsc.VectorSubcoreMesh(core_axis_name="c", subcore_axis_name="s")
    return pl.kernel(
        _sc_count_body,
        out_type=(
            jax.ShapeDtypeStruct((NC, AGG_ROWS, 16), F32),
            jax.ShapeDtypeStruct((4, A_ROWS, 8), F32),
            jax.ShapeDtypeStruct((N2, 16), F32),
        ),
        mesh=mesh,
        compiler_params=pltpu.CompilerParams(use_tc_tiling_on_sc=False),
        scratch_types=(
            pltpu.VMEM_SHARED((AGG_ROWS, 16), F32),
            pltpu.VMEM_SHARED((A_ROWS, 8), F32),
            pltpu.VMEM_SHARED((N2, 16), F32),
            pltpu.VMEM((CPT1, CH), jnp.int32),
            pltpu.VMEM((CH, 8), F32),
            pltpu.VMEM((CH, 16), F32),
            pltpu.VMEM((RPT, 16), F32),
            pltpu.VMEM((A_RPT, 8), F32),
            pltpu.VMEM((2, CH), jnp.int32),
            pltpu.VMEM((1, CH), jnp.int32),
            pltpu.VMEM((1, CH), jnp.int32),
            pltpu.VMEM((1, CH), jnp.int32),
        ),
    )(col3d, etype2d, dst2d, src2d, comp8, zeros16, zeros8, ones16)


def _sc_agg_body(z_lo, z_hi, row3d, col3d, zeros64,
                 agg_lo, agg_hi,
                 agg_sh, rowbuf, colbuf, gbuf, zbuf):
    cid = lax.axis_index("c")
    sid = lax.axis_index("s")
    wid = cid * NS + sid
    base = sid * RPT

    pltpu.sync_copy(zeros64, zbuf)
    pltpu.sync_copy(row3d.at[wid], rowbuf)
    pltpu.sync_copy(col3d.at[wid], colbuf)

    for zref, aggout in ((z_lo, agg_lo), (z_hi, agg_hi)):
        pltpu.sync_copy(zbuf, agg_sh.at[pl.ds(base, RPT)])
        plsc.subcore_barrier()

        def step(j, carry):
            pltpu.sync_copy(zref.at[rowbuf.at[j]], gbuf)
            pltpu.sync_copy(gbuf, agg_sh.at[colbuf.at[j]], add=True)
            return carry

        lax.fori_loop(0, CPT1, step, 0)
        plsc.subcore_barrier()
        pltpu.sync_copy(agg_sh.at[pl.ds(base, RPT)],
                        aggout.at[cid, pl.ds(base, RPT)])
        plsc.subcore_barrier()


def _sc_agg(z_lo, z_hi, row3d, col3d, zeros64):
    mesh = plsc.VectorSubcoreMesh(core_axis_name="c", subcore_axis_name="s")
    return pl.kernel(
        _sc_agg_body,
        out_type=(
            jax.ShapeDtypeStruct((NC, AGG_ROWS, HD), F32),
            jax.ShapeDtypeStruct((NC, AGG_ROWS, HD), F32),
        ),
        mesh=mesh,
        compiler_params=pltpu.CompilerParams(use_tc_tiling_on_sc=False),
        scratch_types=(
            pltpu.VMEM_SHARED((AGG_ROWS, HD), F32),
            pltpu.VMEM((CPT1, CH), jnp.int32),
            pltpu.VMEM((CPT1, CH), jnp.int32),
            pltpu.VMEM((CH, HD), F32),
            pltpu.VMEM((RPT, HD), F32),
        ),
    )(z_lo, z_hi, row3d, col3d, zeros64)


def _tc_prep_body(dm_ref, x_ref, zlo_ref, zhi_ref, dinv_ref):
    dm = dm_ref[...]
    deg = dm[0, :, 0:1] + dm[1, :, 0:1]
    dinv = jnp.where(deg > 0.0, lax.rsqrt(jnp.maximum(deg, 1.0)), 0.0)
    z = x_ref[...] * dinv
    zlo_ref[...] = z[:, :HD]
    zhi_ref[...] = z[:, HD:]
    dinv_ref[...] = dinv


def _tc_prep(degmat, x_g1):
    blk = 1000
    grid = N1 // blk
    return pl.pallas_call(
        _tc_prep_body,
        grid=(grid,),
        in_specs=[
            pl.BlockSpec((NC, blk, 16), lambda i: (0, i, 0)),
            pl.BlockSpec((blk, D), lambda i: (i, 0)),
        ],
        out_specs=[
            pl.BlockSpec((blk, HD), lambda i: (i, 0)),
            pl.BlockSpec((blk, HD), lambda i: (i, 0)),
            pl.BlockSpec((blk, 1), lambda i: (i, 0)),
        ],
        out_shape=[
            jax.ShapeDtypeStruct((N1, HD), F32),
            jax.ShapeDtypeStruct((N1, HD), F32),
            jax.ShapeDtypeStruct((N1, 1), F32),
        ],
    )(degmat, x_g1)


def _tc_out1_body(alo_ref, ahi_ref, dinv_ref, w_ref, b_ref, o_ref):
    a = jnp.concatenate(
        [alo_ref[0] + alo_ref[1], ahi_ref[0] + ahi_ref[1]], axis=1)
    a = a * dinv_ref[...]
    h = jnp.dot(a, w_ref[...], preferred_element_type=F32,
                precision=_HIGH) + b_ref[...]
    h = jnp.maximum(h, 0.0)
    t = h - jnp.max(h, axis=1, keepdims=True)
    o_ref[...] = t - jnp.log(jnp.sum(jnp.exp(t), axis=1, keepdims=True))


def _tc_out1(agg_lo, agg_hi, dinv, W1, b1):
    blk = 1000
    grid = N1 // blk
    return pl.pallas_call(
        _tc_out1_body,
        grid=(grid,),
        in_specs=[
            pl.BlockSpec((NC, blk, HD), lambda i: (0, i, 0)),
            pl.BlockSpec((NC, blk, HD), lambda i: (0, i, 0)),
            pl.BlockSpec((blk, 1), lambda i: (i, 0)),
            pl.BlockSpec((D, D), lambda i: (0, 0)),
            pl.BlockSpec((1, D), lambda i: (0, 0)),
        ],
        out_specs=pl.BlockSpec((blk, D), lambda i: (i, 0)),
        out_shape=jax.ShapeDtypeStruct((N1, D), F32),
    )(agg_lo, agg_hi, dinv, W1, b1)


def _tc_xb_body(x_ref, b_ref, o_ref):
    o_ref[...] = jnp.dot(x_ref[...], b_ref[...], preferred_element_type=F32,
                         precision=_HIGH)


def _tc_xb(x_g2, B2):
    blk = 1024
    grid = (NBP * N2) // blk
    return pl.pallas_call(
        _tc_xb_body,
        grid=(grid,),
        in_specs=[
            pl.BlockSpec((N2, N2), lambda i: (0, 0)),
            pl.BlockSpec((N2, blk), lambda i: (0, i)),
        ],
        out_specs=pl.BlockSpec((N2, blk), lambda i: (0, i)),
        out_shape=jax.ShapeDtypeStruct((N2, NBP * N2), F32),
    )(x_g2, B2)


def _tc_out2_body(a4_ref, xb3_ref, x_ref, root_ref, cnt_ref, b2_ref, o_ref):
    s = jnp.dot(a4_ref[...], xb3_ref[...], preferred_element_type=F32,
                precision=_HIGH)
    cnt = jnp.maximum(cnt_ref[...][:, 0:1], 1.0)
    h = s / cnt + jnp.dot(x_ref[...], root_ref[...],
                          preferred_element_type=F32,
                          precision=_HIGH) + b2_ref[...]
    h = jnp.maximum(h, 0.0)
    t = h - jnp.max(h, axis=1, keepdims=True)
    o_ref[...] = t - jnp.log(jnp.sum(jnp.exp(t), axis=1, keepdims=True))


def _tc_out2(A4, xb3, x_g2, root, cntmat, b2):
    return pl.pallas_call(
        _tc_out2_body,
        out_shape=jax.ShapeDtypeStruct((N2, N2), F32),
    )(A4, xb3, x_g2, root, cntmat, b2)


def kernel(x_g1, edge_index_g1, W1, b1, x_g2, edge_index_g2, edge_type_g2,
           bases, comp, root, b2):
    i32 = jnp.int32
    pad1 = E1P - E1
    row3d = jnp.concatenate(
        [edge_index_g1[0], jnp.zeros((pad1,), i32)]).reshape(NW, CPT1, CH)
    col3d = jnp.concatenate(
        [edge_index_g1[1], jnp.full((pad1,), N1, i32)]).reshape(NW, CPT1, CH)

    etype2d = edge_type_g2.reshape(NS, EPT2)
    src2d = edge_index_g2[0].reshape(NS, EPT2)
    dst2d = edge_index_g2[1].reshape(NS, EPT2)

    comp_pad = jnp.pad(comp, ((0, 0), (0, NBP - NB)))
    comp8 = jnp.concatenate(
        [comp_pad[:, q * 8:(q + 1) * 8] for q in range(4)], axis=0)

    zeros16 = jnp.zeros((RPT, 16), F32)
    zeros8 = jnp.zeros((A_RPT, 8), F32)
    lane = lax.broadcasted_iota(i32, (CH, 16), 1)
    ones16 = jnp.where(lane == 0, 1.0, 0.0).astype(F32)
    zeros64 = jnp.zeros((RPT, HD), F32)

    degmat, a_raw, cntmat = _sc_count(
        col3d, etype2d, dst2d, src2d, comp8, zeros16, zeros8, ones16)

    z_lo, z_hi, dinv = _tc_prep(degmat, x_g1)

    agg_lo, agg_hi = _sc_agg(z_lo, z_hi, row3d, col3d, zeros64)

    out1 = _tc_out1(agg_lo, agg_hi, dinv, W1, b1.reshape(1, D))

    B2 = jnp.pad(bases, ((0, NBP - NB), (0, 0), (0, 0))).transpose(1, 0, 2)
    B2 = B2.reshape(N2, NBP * N2)
    xb2 = _tc_xb(x_g2, B2)
    xb3 = xb2.reshape(NBP * N2, N2)
    A4 = a_raw.transpose(1, 0, 2).reshape(N2, N2 * NBP)
    out2 = _tc_out2(A4, xb3, x_g2, root, cntmat, b2.reshape(1, N2))

    return (out1, out2)


# speedup vs baseline: 8.4414x; 1.0888x over previous
import jax
import jax.numpy as jnp
from jax import lax
from jax.experimental import pallas as pl
from jax.experimental.pallas import tpu as pltpu
from jax.experimental.pallas import tpu_sc as plsc

F32 = jnp.float32

N1 = 10000
E1 = 320000
D = 128
HD = D // 2
N2 = 256
E2 = 2048
NUM_REL = 2048
NB = 30
NBP = 32

NC = 2
NS = 16
NW = NC * NS

CH = 128
CPT1 = 80
EPT1 = CPT1 * CH
E1P = EPT1 * NW

RPT = 632
AGG_ROWS = NS * RPT

A_ROWS = N2 * N2
A_RPT = A_ROWS // NS
EPT2 = E2 // NS

_HIGH = lax.Precision.HIGHEST


def _sc_count_body(col3d, etype2d, dst2d, src2d, comp8, zeros16, zeros8, ones16,
                   deg_out, a_out, cnt_out,
                   degmat_sh, a_sh, cnt_sh,
                   colbuf, compbuf, onesbuf, zbuf, zbuf8, ebuf, dbuf, sbuf,
                   fbuf, dsem):
    cid = lax.axis_index("c")
    sid = lax.axis_index("s")
    wid = cid * NS + sid

    pltpu.sync_copy(zeros16, zbuf)
    pltpu.sync_copy(zeros8, zbuf8)
    pltpu.sync_copy(ones16, onesbuf)
    pltpu.sync_copy(zbuf, degmat_sh.at[pl.ds(sid * RPT, RPT)])

    @pl.when(jnp.logical_and(cid == 0, sid == 0))
    def _zero_cnt():
        pltpu.sync_copy(zbuf.at[pl.ds(0, N2)], cnt_sh)

    pltpu.sync_copy(col3d.at[wid], colbuf)
    pltpu.sync_copy(etype2d.at[sid], ebuf.at[0])
    pltpu.sync_copy(dst2d.at[sid], dbuf.at[0])
    pltpu.sync_copy(src2d.at[sid], sbuf.at[0])

    for k in range(EPT2 // 16):
        sl = pl.ds(k * 16, 16)
        fbuf[0, sl] = dbuf[0, sl] * N2 + sbuf[0, sl]

    plsc.subcore_barrier()

    def deg_step(i, carry):
        for k in range(8):
            pltpu.async_copy(onesbuf, degmat_sh.at[colbuf.at[i * 8 + k]],
                             dsem, add=True)
        for k in range(8):
            pltpu.make_async_copy(
                onesbuf, degmat_sh.at[colbuf.at[i * 8 + k]], dsem).wait()
        return carry

    lax.fori_loop(0, CPT1 // 8, deg_step, 0)

    @pl.when(cid == 0)
    def _cnt():
        pltpu.sync_copy(onesbuf, cnt_sh.at[dbuf.at[0]], add=True)

    for g in range(2):
        gidx = cid * 2 + g
        for k in range(EPT2 // 16):
            sl = pl.ds(k * 16, 16)
            ebuf[1, sl] = ebuf[0, sl] + gidx * NUM_REL
        pltpu.sync_copy(zbuf8, a_sh.at[pl.ds(sid * A_RPT, A_RPT)])
        plsc.subcore_barrier()
        pltpu.sync_copy(comp8.at[ebuf.at[1]], compbuf)
        pltpu.sync_copy(compbuf, a_sh.at[fbuf.at[0]], add=True)
        plsc.subcore_barrier()
        pltpu.sync_copy(a_sh.at[pl.ds(sid * A_RPT, A_RPT)],
                        a_out.at[gidx, pl.ds(sid * A_RPT, A_RPT)])

    plsc.subcore_barrier()
    pltpu.sync_copy(degmat_sh.at[pl.ds(sid * RPT, RPT)],
                    deg_out.at[cid, pl.ds(sid * RPT, RPT)])

    @pl.when(cid == 0)
    def _dump_cnt():
        pltpu.sync_copy(cnt_sh.at[pl.ds(sid * 16, 16)],
                        cnt_out.at[pl.ds(sid * 16, 16)])


def _sc_count(col3d, etype2d, dst2d, src2d, comp8, zeros16, zeros8, ones16):
    mesh = plsc.VectorSubcoreMesh(core_axis_name="c", subcore_axis_name="s")
    return pl.kernel(
        _sc_count_body,
        out_type=(
            jax.ShapeDtypeStruct((NC, AGG_ROWS, 16), F32),
            jax.ShapeDtypeStruct((4, A_ROWS, 8), F32),
            jax.ShapeDtypeStruct((N2, 16), F32),
        ),
        mesh=mesh,
        compiler_params=pltpu.CompilerParams(use_tc_tiling_on_sc=False),
        scratch_types=(
            pltpu.VMEM_SHARED((AGG_ROWS, 16), F32),
            pltpu.VMEM_SHARED((A_ROWS, 8), F32),
            pltpu.VMEM_SHARED((N2, 16), F32),
            pltpu.VMEM((CPT1, CH), jnp.int32),
            pltpu.VMEM((CH, 8), F32),
            pltpu.VMEM((CH, 16), F32),
            pltpu.VMEM((RPT, 16), F32),
            pltpu.VMEM((A_RPT, 8), F32),
            pltpu.VMEM((2, CH), jnp.int32),
            pltpu.VMEM((1, CH), jnp.int32),
            pltpu.VMEM((1, CH), jnp.int32),
            pltpu.VMEM((1, CH), jnp.int32),
            pltpu.SemaphoreType.DMA,
        ),
    )(col3d, etype2d, dst2d, src2d, comp8, zeros16, zeros8, ones16)


G = 2
NGP = CPT1 // (2 * G)


def _sc_agg_body(z_lo, z_hi, row3d, col3d, zeros64,
                 agg_lo, agg_hi,
                 agg_sh, rowbuf, colbuf, gbufs, zbuf, gsem, ssem):
    cid = lax.axis_index("c")
    sid = lax.axis_index("s")
    wid = cid * NS + sid
    base = sid * RPT

    pltpu.sync_copy(zeros64, zbuf)
    pltpu.sync_copy(row3d.at[wid], rowbuf)
    pltpu.sync_copy(col3d.at[wid], colbuf)

    for zref, aggout in ((z_lo, agg_lo), (z_hi, agg_hi)):
        for k in range(4):
            pltpu.sync_copy(zbuf, agg_sh.at[pl.ds(base + k * CH, CH)])
        pltpu.sync_copy(zbuf.at[pl.ds(0, RPT - 4 * CH)],
                        agg_sh.at[pl.ds(base + 4 * CH, RPT - 4 * CH)])
        plsc.subcore_barrier()

        def fire_gathers(j0, bufset):
            for k in range(G):
                pltpu.async_copy(zref.at[rowbuf.at[j0 + k]],
                                 gbufs.at[bufset * G + k], gsem)

        def drain_gathers(bufset):
            for k in range(G):
                pltpu.make_async_copy(zref.at[rowbuf.at[0]],
                                      gbufs.at[bufset * G + k], gsem).wait()

        def fire_scatters(j0, bufset):
            for k in range(G):
                pltpu.async_copy(gbufs.at[bufset * G + k],
                                 agg_sh.at[colbuf.at[j0 + k]], ssem, add=True)

        def drain_scatters(bufset):
            for k in range(G):
                pltpu.make_async_copy(gbufs.at[bufset * G + k],
                                      agg_sh.at[colbuf.at[0]], ssem).wait()

        fire_gathers(0, 0)

        def step(i, carry):
            j0 = 2 * G * i
            drain_gathers(0)

            @pl.when(i > 0)
            def _():
                drain_scatters(1)

            fire_gathers(j0 + G, 1)
            fire_scatters(j0, 0)
            drain_gathers(1)
            drain_scatters(0)

            @pl.when(i < NGP - 1)
            def _():
                fire_gathers(j0 + 2 * G, 0)

            fire_scatters(j0 + G, 1)
            return carry

        lax.fori_loop(0, NGP, step, 0)
        drain_scatters(1)
        plsc.subcore_barrier()
        pltpu.sync_copy(agg_sh.at[pl.ds(base, RPT)],
                        aggout.at[cid, pl.ds(base, RPT)])
        plsc.subcore_barrier()


def _sc_agg(z_lo, z_hi, row3d, col3d, zeros64):
    mesh = plsc.VectorSubcoreMesh(core_axis_name="c", subcore_axis_name="s")
    return pl.kernel(
        _sc_agg_body,
        out_type=(
            jax.ShapeDtypeStruct((NC, AGG_ROWS, HD), F32),
            jax.ShapeDtypeStruct((NC, AGG_ROWS, HD), F32),
        ),
        mesh=mesh,
        compiler_params=pltpu.CompilerParams(use_tc_tiling_on_sc=False),
        scratch_types=(
            pltpu.VMEM_SHARED((AGG_ROWS, HD), F32),
            pltpu.VMEM((CPT1, CH), jnp.int32),
            pltpu.VMEM((CPT1, CH), jnp.int32),
            pltpu.VMEM((2 * G, CH, HD), F32),
            pltpu.VMEM((CH, HD), F32),
            pltpu.SemaphoreType.DMA,
            pltpu.SemaphoreType.DMA,
        ),
    )(z_lo, z_hi, row3d, col3d, zeros64)


def _tc_prep_body(dm_ref, x_ref, zlo_ref, zhi_ref, dinv_ref):
    dm = dm_ref[...]
    deg = dm[0, :, 0:1] + dm[1, :, 0:1]
    dinv = jnp.where(deg > 0.0, lax.rsqrt(jnp.maximum(deg, 1.0)), 0.0)
    z = x_ref[...] * dinv
    zlo_ref[...] = z[:, :HD]
    zhi_ref[...] = z[:, HD:]
    dinv_ref[...] = dinv


def _tc_prep(degmat, x_g1):
    blk = 1000
    grid = N1 // blk
    return pl.pallas_call(
        _tc_prep_body,
        grid=(grid,),
        in_specs=[
            pl.BlockSpec((NC, blk, 16), lambda i: (0, i, 0)),
            pl.BlockSpec((blk, D), lambda i: (i, 0)),
        ],
        out_specs=[
            pl.BlockSpec((blk, HD), lambda i: (i, 0)),
            pl.BlockSpec((blk, HD), lambda i: (i, 0)),
            pl.BlockSpec((blk, 1), lambda i: (i, 0)),
        ],
        out_shape=[
            jax.ShapeDtypeStruct((N1, HD), F32),
            jax.ShapeDtypeStruct((N1, HD), F32),
            jax.ShapeDtypeStruct((N1, 1), F32),
        ],
    )(degmat, x_g1)


def _tc_out1_body(alo_ref, ahi_ref, dinv_ref, w_ref, b_ref, o_ref):
    a = jnp.concatenate(
        [alo_ref[0] + alo_ref[1], ahi_ref[0] + ahi_ref[1]], axis=1)
    a = a * dinv_ref[...]
    h = jnp.dot(a, w_ref[...], preferred_element_type=F32,
                precision=_HIGH) + b_ref[...]
    h = jnp.maximum(h, 0.0)
    t = h - jnp.max(h, axis=1, keepdims=True)
    o_ref[...] = t - jnp.log(jnp.sum(jnp.exp(t), axis=1, keepdims=True))


def _tc_out1(agg_lo, agg_hi, dinv, W1, b1):
    blk = 1000
    grid = N1 // blk
    return pl.pallas_call(
        _tc_out1_body,
        grid=(grid,),
        in_specs=[
            pl.BlockSpec((NC, blk, HD), lambda i: (0, i, 0)),
            pl.BlockSpec((NC, blk, HD), lambda i: (0, i, 0)),
            pl.BlockSpec((blk, 1), lambda i: (i, 0)),
            pl.BlockSpec((D, D), lambda i: (0, 0)),
            pl.BlockSpec((1, D), lambda i: (0, 0)),
        ],
        out_specs=pl.BlockSpec((blk, D), lambda i: (i, 0)),
        out_shape=jax.ShapeDtypeStruct((N1, D), F32),
    )(agg_lo, agg_hi, dinv, W1, b1)


def _tc_xb_body(x_ref, b_ref, o_ref):
    o_ref[...] = jnp.dot(x_ref[...], b_ref[...], preferred_element_type=F32,
                         precision=_HIGH)


def _tc_xb(x_g2, B2):
    blk = 1024
    grid = (NBP * N2) // blk
    return pl.pallas_call(
        _tc_xb_body,
        grid=(grid,),
        in_specs=[
            pl.BlockSpec((N2, N2), lambda i: (0, 0)),
            pl.BlockSpec((N2, blk), lambda i: (0, i)),
        ],
        out_specs=pl.BlockSpec((N2, blk), lambda i: (0, i)),
        out_shape=jax.ShapeDtypeStruct((N2, NBP * N2), F32),
    )(x_g2, B2)


def _tc_out2_body(a4_ref, xb3_ref, x_ref, root_ref, cnt_ref, b2_ref, o_ref):
    s = jnp.dot(a4_ref[...], xb3_ref[...], preferred_element_type=F32,
                precision=_HIGH)
    cnt = jnp.maximum(cnt_ref[...][:, 0:1], 1.0)
    h = s / cnt + jnp.dot(x_ref[...], root_ref[...],
                          preferred_element_type=F32,
                          precision=_HIGH) + b2_ref[...]
    h = jnp.maximum(h, 0.0)
    t = h - jnp.max(h, axis=1, keepdims=True)
    o_ref[...] = t - jnp.log(jnp.sum(jnp.exp(t), axis=1, keepdims=True))


def _tc_out2(A4, xb3, x_g2, root, cntmat, b2):
    return pl.pallas_call(
        _tc_out2_body,
        out_shape=jax.ShapeDtypeStruct((N2, N2), F32),
    )(A4, xb3, x_g2, root, cntmat, b2)


def kernel(x_g1, edge_index_g1, W1, b1, x_g2, edge_index_g2, edge_type_g2,
           bases, comp, root, b2):
    i32 = jnp.int32
    pad1 = E1P - E1
    row3d = jnp.concatenate(
        [edge_index_g1[0], jnp.zeros((pad1,), i32)]).reshape(NW, CPT1, CH)
    col3d = jnp.concatenate(
        [edge_index_g1[1], jnp.full((pad1,), N1, i32)]).reshape(NW, CPT1, CH)

    etype2d = edge_type_g2.reshape(NS, EPT2)
    src2d = edge_index_g2[0].reshape(NS, EPT2)
    dst2d = edge_index_g2[1].reshape(NS, EPT2)

    comp_pad = jnp.pad(comp, ((0, 0), (0, NBP - NB)))
    comp8 = jnp.concatenate(
        [comp_pad[:, q * 8:(q + 1) * 8] for q in range(4)], axis=0)

    zeros16 = jnp.zeros((RPT, 16), F32)
    zeros8 = jnp.zeros((A_RPT, 8), F32)
    lane = lax.broadcasted_iota(i32, (CH, 16), 1)
    ones16 = jnp.where(lane == 0, 1.0, 0.0).astype(F32)
    zeros64 = jnp.zeros((CH, HD), F32)

    degmat, a_raw, cntmat = _sc_count(
        col3d, etype2d, dst2d, src2d, comp8, zeros16, zeros8, ones16)

    z_lo, z_hi, dinv = _tc_prep(degmat, x_g1)

    agg_lo, agg_hi = _sc_agg(z_lo, z_hi, row3d, col3d, zeros64)

    out1 = _tc_out1(agg_lo, agg_hi, dinv, W1, b1.reshape(1, D))

    B2 = jnp.pad(bases, ((0, NBP - NB), (0, 0), (0, 0))).transpose(1, 0, 2)
    B2 = B2.reshape(N2, NBP * N2)
    xb2 = _tc_xb(x_g2, B2)
    xb3 = xb2.reshape(NBP * N2, N2)
    A4 = a_raw.transpose(1, 0, 2).reshape(N2, N2 * NBP)
    out2 = _tc_out2(A4, xb3, x_g2, root, cntmat, b2.reshape(1, N2))

    return (out1, out2)


# speedup vs baseline: 8.7153x; 1.0325x over previous
import jax
import jax.numpy as jnp
from jax import lax
from jax.experimental import pallas as pl
from jax.experimental.pallas import tpu as pltpu
from jax.experimental.pallas import tpu_sc as plsc

F32 = jnp.float32

N1 = 10000
E1 = 320000
D = 128
HD = D // 2
N2 = 256
E2 = 2048
NUM_REL = 2048
NB = 30
NBP = 32

NC = 2
NS = 16
NW = NC * NS

CH = 128
CPT1 = 80
EPT1 = CPT1 * CH
E1P = EPT1 * NW

RPT = 632
AGG_ROWS = NS * RPT

A_ROWS = N2 * N2
A_RPT = A_ROWS // NS
EPT2 = E2 // NS

_HIGH = lax.Precision.HIGHEST


def _sc_count_body(col3d, etype2d, dst2d, src2d, comp8, zeros16, zeros8, ones16,
                   deg_out, a_out, cnt_out,
                   degmat_sh, a_sh, cnt_sh,
                   colbuf, compbuf, onesbuf, zbuf, zbuf8, ebuf, dbuf, sbuf,
                   fbuf, dsem):
    cid = lax.axis_index("c")
    sid = lax.axis_index("s")
    wid = cid * NS + sid

    pltpu.sync_copy(zeros16, zbuf)
    pltpu.sync_copy(zeros8, zbuf8)
    pltpu.sync_copy(ones16, onesbuf)
    pltpu.sync_copy(zbuf, degmat_sh.at[pl.ds(sid * RPT, RPT)])

    @pl.when(jnp.logical_and(cid == 0, sid == 0))
    def _zero_cnt():
        pltpu.sync_copy(zbuf.at[pl.ds(0, N2)], cnt_sh)

    pltpu.sync_copy(col3d.at[wid], colbuf)
    pltpu.sync_copy(etype2d.at[sid], ebuf.at[0])
    pltpu.sync_copy(dst2d.at[sid], dbuf.at[0])
    pltpu.sync_copy(src2d.at[sid], sbuf.at[0])

    for k in range(EPT2 // 16):
        sl = pl.ds(k * 16, 16)
        fbuf[0, sl] = dbuf[0, sl] * N2 + sbuf[0, sl]

    plsc.subcore_barrier()

    def deg_step(i, carry):
        for k in range(8):
            pltpu.async_copy(onesbuf, degmat_sh.at[colbuf.at[i * 8 + k]],
                             dsem, add=True)
        for k in range(8):
            pltpu.make_async_copy(
                onesbuf, degmat_sh.at[colbuf.at[i * 8 + k]], dsem).wait()
        return carry

    lax.fori_loop(0, CPT1 // 8, deg_step, 0)

    @pl.when(cid == 0)
    def _cnt():
        pltpu.sync_copy(onesbuf, cnt_sh.at[dbuf.at[0]], add=True)

    for g in range(2):
        gidx = cid * 2 + g
        for k in range(EPT2 // 16):
            sl = pl.ds(k * 16, 16)
            ebuf[1, sl] = ebuf[0, sl] + gidx * NUM_REL
        pltpu.sync_copy(zbuf8, a_sh.at[pl.ds(sid * A_RPT, A_RPT)])
        plsc.subcore_barrier()
        pltpu.sync_copy(comp8.at[ebuf.at[1]], compbuf)
        pltpu.sync_copy(compbuf, a_sh.at[fbuf.at[0]], add=True)
        plsc.subcore_barrier()
        pltpu.sync_copy(a_sh.at[pl.ds(sid * A_RPT, A_RPT)],
                        a_out.at[pl.ds(sid * A_RPT, A_RPT),
                                 pl.ds(gidx * 8, 8)])

    plsc.subcore_barrier()
    pltpu.sync_copy(degmat_sh.at[pl.ds(sid * RPT, RPT)],
                    deg_out.at[cid, pl.ds(sid * RPT, RPT)])

    @pl.when(cid == 0)
    def _dump_cnt():
        pltpu.sync_copy(cnt_sh.at[pl.ds(sid * 16, 16)],
                        cnt_out.at[pl.ds(sid * 16, 16)])


def _sc_count(col3d, etype2d, dst2d, src2d, comp8, zeros16, zeros8, ones16):
    mesh = plsc.VectorSubcoreMesh(core_axis_name="c", subcore_axis_name="s")
    return pl.kernel(
        _sc_count_body,
        out_type=(
            jax.ShapeDtypeStruct((NC, AGG_ROWS, 16), F32),
            jax.ShapeDtypeStruct((A_ROWS, NBP), F32),
            jax.ShapeDtypeStruct((N2, 16), F32),
        ),
        mesh=mesh,
        compiler_params=pltpu.CompilerParams(use_tc_tiling_on_sc=False),
        scratch_types=(
            pltpu.VMEM_SHARED((AGG_ROWS, 16), F32),
            pltpu.VMEM_SHARED((A_ROWS, 8), F32),
            pltpu.VMEM_SHARED((N2, 16), F32),
            pltpu.VMEM((CPT1, CH), jnp.int32),
            pltpu.VMEM((CH, 8), F32),
            pltpu.VMEM((CH, 16), F32),
            pltpu.VMEM((RPT, 16), F32),
            pltpu.VMEM((A_RPT, 8), F32),
            pltpu.VMEM((2, CH), jnp.int32),
            pltpu.VMEM((1, CH), jnp.int32),
            pltpu.VMEM((1, CH), jnp.int32),
            pltpu.VMEM((1, CH), jnp.int32),
            pltpu.SemaphoreType.DMA,
        ),
    )(col3d, etype2d, dst2d, src2d, comp8, zeros16, zeros8, ones16)


G = 2
NGP = CPT1 // (2 * G)


def _sc_agg_body(z_lo, z_hi, row3d, col3d, zeros64,
                 agg_lo, agg_hi,
                 agg_sh, rowbuf, colbuf, gbufs, zbuf, gsem, ssem):
    cid = lax.axis_index("c")
    sid = lax.axis_index("s")
    wid = cid * NS + sid
    base = sid * RPT

    pltpu.sync_copy(zeros64, zbuf)
    pltpu.sync_copy(row3d.at[wid], rowbuf)
    pltpu.sync_copy(col3d.at[wid], colbuf)

    for zref, aggout in ((z_lo, agg_lo), (z_hi, agg_hi)):
        for k in range(4):
            pltpu.sync_copy(zbuf, agg_sh.at[pl.ds(base + k * CH, CH)])
        pltpu.sync_copy(zbuf.at[pl.ds(0, RPT - 4 * CH)],
                        agg_sh.at[pl.ds(base + 4 * CH, RPT - 4 * CH)])
        plsc.subcore_barrier()

        def fire_gathers(j0, bufset):
            for k in range(G):
                pltpu.async_copy(zref.at[rowbuf.at[j0 + k]],
                                 gbufs.at[bufset * G + k], gsem)

        def drain_gathers(bufset):
            for k in range(G):
                pltpu.make_async_copy(zref.at[rowbuf.at[0]],
                                      gbufs.at[bufset * G + k], gsem).wait()

        def fire_scatters(j0, bufset):
            for k in range(G):
                pltpu.async_copy(gbufs.at[bufset * G + k],
                                 agg_sh.at[colbuf.at[j0 + k]], ssem, add=True)

        def drain_scatters(bufset):
            for k in range(G):
                pltpu.make_async_copy(gbufs.at[bufset * G + k],
                                      agg_sh.at[colbuf.at[0]], ssem).wait()

        fire_gathers(0, 0)

        def step(i, carry):
            j0 = 2 * G * i
            drain_gathers(0)

            @pl.when(i > 0)
            def _():
                drain_scatters(1)

            fire_gathers(j0 + G, 1)
            fire_scatters(j0, 0)
            drain_gathers(1)
            drain_scatters(0)

            @pl.when(i < NGP - 1)
            def _():
                fire_gathers(j0 + 2 * G, 0)

            fire_scatters(j0 + G, 1)
            return carry

        lax.fori_loop(0, NGP, step, 0)
        drain_scatters(1)
        plsc.subcore_barrier()
        pltpu.sync_copy(agg_sh.at[pl.ds(base, RPT)],
                        aggout.at[cid, pl.ds(base, RPT)])
        plsc.subcore_barrier()


def _sc_agg(z_lo, z_hi, row3d, col3d, zeros64):
    mesh = plsc.VectorSubcoreMesh(core_axis_name="c", subcore_axis_name="s")
    return pl.kernel(
        _sc_agg_body,
        out_type=(
            jax.ShapeDtypeStruct((NC, AGG_ROWS, HD), F32),
            jax.ShapeDtypeStruct((NC, AGG_ROWS, HD), F32),
        ),
        mesh=mesh,
        compiler_params=pltpu.CompilerParams(use_tc_tiling_on_sc=False),
        scratch_types=(
            pltpu.VMEM_SHARED((AGG_ROWS, HD), F32),
            pltpu.VMEM((CPT1, CH), jnp.int32),
            pltpu.VMEM((CPT1, CH), jnp.int32),
            pltpu.VMEM((2 * G, CH, HD), F32),
            pltpu.VMEM((CH, HD), F32),
            pltpu.SemaphoreType.DMA,
            pltpu.SemaphoreType.DMA,
        ),
    )(z_lo, z_hi, row3d, col3d, zeros64)


def _tc_prep_body(dm_ref, x_ref, zlo_ref, zhi_ref, dinv_ref):
    dm = dm_ref[...]
    deg = dm[0, :, 0:1] + dm[1, :, 0:1]
    dinv = jnp.where(deg > 0.0, lax.rsqrt(jnp.maximum(deg, 1.0)), 0.0)
    z = x_ref[...] * dinv
    zlo_ref[...] = z[:, :HD]
    zhi_ref[...] = z[:, HD:]
    dinv_ref[...] = dinv


def _tc_prep(degmat, x_g1):
    blk = 1000
    grid = N1 // blk
    return pl.pallas_call(
        _tc_prep_body,
        grid=(grid,),
        in_specs=[
            pl.BlockSpec((NC, blk, 16), lambda i: (0, i, 0)),
            pl.BlockSpec((blk, D), lambda i: (i, 0)),
        ],
        out_specs=[
            pl.BlockSpec((blk, HD), lambda i: (i, 0)),
            pl.BlockSpec((blk, HD), lambda i: (i, 0)),
            pl.BlockSpec((blk, 1), lambda i: (i, 0)),
        ],
        out_shape=[
            jax.ShapeDtypeStruct((N1, HD), F32),
            jax.ShapeDtypeStruct((N1, HD), F32),
            jax.ShapeDtypeStruct((N1, 1), F32),
        ],
    )(degmat, x_g1)


def _tc_out1_body(alo_ref, ahi_ref, dinv_ref, w_ref, b_ref, o_ref):
    a = jnp.concatenate(
        [alo_ref[0] + alo_ref[1], ahi_ref[0] + ahi_ref[1]], axis=1)
    a = a * dinv_ref[...]
    h = jnp.dot(a, w_ref[...], preferred_element_type=F32,
                precision=_HIGH) + b_ref[...]
    h = jnp.maximum(h, 0.0)
    t = h - jnp.max(h, axis=1, keepdims=True)
    o_ref[...] = t - jnp.log(jnp.sum(jnp.exp(t), axis=1, keepdims=True))


def _tc_out1(agg_lo, agg_hi, dinv, W1, b1):
    blk = 1000
    grid = N1 // blk
    return pl.pallas_call(
        _tc_out1_body,
        grid=(grid,),
        in_specs=[
            pl.BlockSpec((NC, blk, HD), lambda i: (0, i, 0)),
            pl.BlockSpec((NC, blk, HD), lambda i: (0, i, 0)),
            pl.BlockSpec((blk, 1), lambda i: (i, 0)),
            pl.BlockSpec((D, D), lambda i: (0, 0)),
            pl.BlockSpec((1, D), lambda i: (0, 0)),
        ],
        out_specs=pl.BlockSpec((blk, D), lambda i: (i, 0)),
        out_shape=jax.ShapeDtypeStruct((N1, D), F32),
    )(agg_lo, agg_hi, dinv, W1, b1)


def _tc_xb_body(x_ref, b_ref, o_ref):
    gidx = pl.program_id(0)
    x = x_ref[...]
    parts = []
    for bb in range(8):
        r = jnp.dot(x, b_ref[bb], preferred_element_type=F32,
                    precision=_HIGH)
        r = jnp.where(gidx * 8 + bb < NB, r, 0.0)
        parts.append(r[:, None, :])
    o_ref[...] = jnp.concatenate(parts, axis=1)


def _tc_xb(x_g2, bases):
    return pl.pallas_call(
        _tc_xb_body,
        grid=(NBP // 8,),
        in_specs=[
            pl.BlockSpec((N2, N2), lambda i: (0, 0)),
            pl.BlockSpec((8, N2, N2), lambda i: (i, 0, 0)),
        ],
        out_specs=pl.BlockSpec((N2, 8, N2), lambda i: (0, i, 0)),
        out_shape=jax.ShapeDtypeStruct((N2, NBP, N2), F32),
    )(x_g2, bases)


def _tc_out2_body(a4_ref, xb3_ref, x_ref, root_ref, cnt_ref, b2_ref, o_ref):
    s = jnp.dot(a4_ref[...], xb3_ref[...], preferred_element_type=F32,
                precision=_HIGH)
    cnt = jnp.maximum(cnt_ref[...][:, 0:1], 1.0)
    h = s / cnt + jnp.dot(x_ref[...], root_ref[...],
                          preferred_element_type=F32,
                          precision=_HIGH) + b2_ref[...]
    h = jnp.maximum(h, 0.0)
    t = h - jnp.max(h, axis=1, keepdims=True)
    o_ref[...] = t - jnp.log(jnp.sum(jnp.exp(t), axis=1, keepdims=True))


def _tc_out2(A4, xb3, x_g2, root, cntmat, b2):
    return pl.pallas_call(
        _tc_out2_body,
        out_shape=jax.ShapeDtypeStruct((N2, N2), F32),
    )(A4, xb3, x_g2, root, cntmat, b2)


def kernel(x_g1, edge_index_g1, W1, b1, x_g2, edge_index_g2, edge_type_g2,
           bases, comp, root, b2):
    i32 = jnp.int32
    pad1 = E1P - E1
    row3d = jnp.concatenate(
        [edge_index_g1[0], jnp.zeros((pad1,), i32)]).reshape(NW, CPT1, CH)
    col3d = jnp.concatenate(
        [edge_index_g1[1],
         N1 + jnp.arange(pad1, dtype=i32) % (AGG_ROWS - N1)]
    ).reshape(NW, CPT1, CH)

    etype2d = edge_type_g2.reshape(NS, EPT2)
    src2d = edge_index_g2[0].reshape(NS, EPT2)
    dst2d = edge_index_g2[1].reshape(NS, EPT2)

    comp_pad = jnp.pad(comp, ((0, 0), (0, NBP - NB)))
    comp8 = jnp.concatenate(
        [comp_pad[:, q * 8:(q + 1) * 8] for q in range(4)], axis=0)

    zeros16 = jnp.zeros((RPT, 16), F32)
    zeros8 = jnp.zeros((A_RPT, 8), F32)
    lane = lax.broadcasted_iota(i32, (CH, 16), 1)
    ones16 = jnp.where(lane == 0, 1.0, 0.0).astype(F32)
    zeros64 = jnp.zeros((CH, HD), F32)

    degmat, a_raw, cntmat = _sc_count(
        col3d, etype2d, dst2d, src2d, comp8, zeros16, zeros8, ones16)

    z_lo, z_hi, dinv = _tc_prep(degmat, x_g1)

    agg_lo, agg_hi = _sc_agg(z_lo, z_hi, row3d, col3d, zeros64)

    out1 = _tc_out1(agg_lo, agg_hi, dinv, W1, b1.reshape(1, D))

    xb2 = _tc_xb(x_g2, bases)
    xb3 = xb2.reshape(NBP * N2, N2)
    A4 = a_raw.reshape(N2, N2 * NBP)
    out2 = _tc_out2(A4, xb3, x_g2, root, cntmat, b2.reshape(1, N2))

    return (out1, out2)


# speedup vs baseline: 9.4351x; 1.0826x over previous
import jax
import jax.numpy as jnp
from jax import lax
from jax.experimental import pallas as pl
from jax.experimental.pallas import tpu as pltpu
from jax.experimental.pallas import tpu_sc as plsc

F32 = jnp.float32

N1 = 10000
E1 = 320000
D = 128
HD = D // 2
N2 = 256
E2 = 2048
NUM_REL = 2048
NB = 30
NBP = 32

NC = 2
NS = 16
NW = NC * NS

CH = 128
CPT1 = 80
EPT1 = CPT1 * CH
E1P = EPT1 * NW

RPT = 632
AGG_ROWS = NS * RPT

A_ROWS = N2 * N2
A_RPT = A_ROWS // NS
EPT2 = E2 // NS

_HIGH = lax.Precision.HIGHEST


def _sc_count_body(col3d, etype2d, dst2d, src2d, comp8, zeros16, zeros8, ones16,
                   deg_out, a_out, cnt_out,
                   degmat_sh, a_sh, cnt_sh,
                   colbuf, compbuf, onesbuf, zbuf, zbuf8, ebuf, dbuf, sbuf,
                   fbuf, dsem):
    cid = lax.axis_index("c")
    sid = lax.axis_index("s")
    wid = cid * NS + sid

    pltpu.sync_copy(zeros16, zbuf)
    pltpu.sync_copy(zeros8, zbuf8)
    pltpu.sync_copy(ones16, onesbuf)
    pltpu.sync_copy(zbuf, degmat_sh.at[pl.ds(sid * RPT, RPT)])

    @pl.when(jnp.logical_and(cid == 0, sid == 0))
    def _zero_cnt():
        pltpu.sync_copy(zbuf.at[pl.ds(0, N2)], cnt_sh)

    pltpu.sync_copy(col3d.at[pl.ds(wid * CPT1, CPT1)], colbuf)
    pltpu.sync_copy(etype2d.at[sid], ebuf.at[0])
    pltpu.sync_copy(dst2d.at[sid], dbuf.at[0])
    pltpu.sync_copy(src2d.at[sid], sbuf.at[0])

    for k in range(EPT2 // 16):
        sl = pl.ds(k * 16, 16)
        fbuf[0, sl] = dbuf[0, sl] * N2 + sbuf[0, sl]

    plsc.subcore_barrier()

    def deg_step(i, carry):
        for k in range(8):
            pltpu.async_copy(onesbuf, degmat_sh.at[colbuf.at[i * 8 + k]],
                             dsem, add=True)
        for k in range(8):
            pltpu.make_async_copy(
                onesbuf, degmat_sh.at[colbuf.at[i * 8 + k]], dsem).wait()
        return carry

    lax.fori_loop(0, CPT1 // 8, deg_step, 0)

    @pl.when(cid == 0)
    def _cnt():
        pltpu.sync_copy(onesbuf, cnt_sh.at[dbuf.at[0]], add=True)

    for g in range(2):
        gidx = cid * 2 + g
        for k in range(EPT2 // 16):
            sl = pl.ds(k * 16, 16)
            ebuf[1, sl] = ebuf[0, sl] + gidx * NUM_REL
        pltpu.sync_copy(zbuf8, a_sh.at[pl.ds(sid * A_RPT, A_RPT)])
        plsc.subcore_barrier()
        pltpu.sync_copy(comp8.at[ebuf.at[1]], compbuf)
        pltpu.sync_copy(compbuf, a_sh.at[fbuf.at[0]], add=True)
        plsc.subcore_barrier()
        pltpu.sync_copy(a_sh.at[pl.ds(sid * A_RPT, A_RPT)],
                        a_out.at[pl.ds(sid * A_RPT, A_RPT),
                                 pl.ds(gidx * 8, 8)])

    plsc.subcore_barrier()
    pltpu.sync_copy(degmat_sh.at[pl.ds(sid * RPT, RPT)],
                    deg_out.at[cid, pl.ds(sid * RPT, RPT)])

    @pl.when(cid == 0)
    def _dump_cnt():
        pltpu.sync_copy(cnt_sh.at[pl.ds(sid * 16, 16)],
                        cnt_out.at[pl.ds(sid * 16, 16)])


def _sc_count(col3d, etype2d, dst2d, src2d, comp8, zeros16, zeros8, ones16):
    mesh = plsc.VectorSubcoreMesh(core_axis_name="c", subcore_axis_name="s")
    return pl.kernel(
        _sc_count_body,
        out_type=(
            jax.ShapeDtypeStruct((NC, AGG_ROWS, 16), F32),
            jax.ShapeDtypeStruct((A_ROWS, NBP), F32),
            jax.ShapeDtypeStruct((N2, 16), F32),
        ),
        mesh=mesh,
        compiler_params=pltpu.CompilerParams(use_tc_tiling_on_sc=False),
        scratch_types=(
            pltpu.VMEM_SHARED((AGG_ROWS, 16), F32),
            pltpu.VMEM_SHARED((A_ROWS, 8), F32),
            pltpu.VMEM_SHARED((N2, 16), F32),
            pltpu.VMEM((CPT1, CH), jnp.int32),
            pltpu.VMEM((CH, 8), F32),
            pltpu.VMEM((CH, 16), F32),
            pltpu.VMEM((RPT, 16), F32),
            pltpu.VMEM((A_RPT, 8), F32),
            pltpu.VMEM((2, CH), jnp.int32),
            pltpu.VMEM((1, CH), jnp.int32),
            pltpu.VMEM((1, CH), jnp.int32),
            pltpu.VMEM((1, CH), jnp.int32),
            pltpu.SemaphoreType.DMA,
        ),
    )(col3d, etype2d, dst2d, src2d, comp8, zeros16, zeros8, ones16)


G = 2
CPT_A = 124
CPT_B = 36


def _sc_agg_body(z_lo, z_hi, row2d, col2d, zeros64,
                 agg_lo, agg_hi,
                 agg_sh, rowbuf, colbuf, gbufs, zbuf, gsem, ssem):
    cid = lax.axis_index("c")
    sid = lax.axis_index("s")
    base = sid * RPT

    pltpu.sync_copy(zeros64, zbuf)

    @pl.when(cid == 0)
    def _stage_a():
        pltpu.sync_copy(row2d.at[pl.ds(sid * CPT_A, CPT_A)], rowbuf)
        pltpu.sync_copy(col2d.at[pl.ds(sid * CPT_A, CPT_A)], colbuf)

    @pl.when(cid == 1)
    def _stage_b():
        pltpu.sync_copy(row2d.at[pl.ds(NS * CPT_A + sid * CPT_B, CPT_B)],
                        rowbuf.at[pl.ds(0, CPT_B)])
        pltpu.sync_copy(col2d.at[pl.ds(NS * CPT_A + sid * CPT_B, CPT_B)],
                        colbuf.at[pl.ds(0, CPT_B)])

    def run_pipeline(zref, cpt):
        ngp = cpt // (2 * G)

        def fire_gathers(j0, bufset):
            for k in range(G):
                pltpu.async_copy(zref.at[rowbuf.at[j0 + k]],
                                 gbufs.at[bufset * G + k], gsem)

        def drain_gathers(bufset):
            for k in range(G):
                pltpu.make_async_copy(zref.at[rowbuf.at[0]],
                                      gbufs.at[bufset * G + k], gsem).wait()

        def fire_scatters(j0, bufset):
            for k in range(G):
                pltpu.async_copy(gbufs.at[bufset * G + k],
                                 agg_sh.at[colbuf.at[j0 + k]], ssem, add=True)

        def drain_scatters(bufset):
            for k in range(G):
                pltpu.make_async_copy(gbufs.at[bufset * G + k],
                                      agg_sh.at[colbuf.at[0]], ssem).wait()

        fire_gathers(0, 0)

        def step(i, carry):
            j0 = 2 * G * i
            drain_gathers(0)

            @pl.when(i > 0)
            def _():
                drain_scatters(1)

            fire_gathers(j0 + G, 1)
            fire_scatters(j0, 0)
            drain_gathers(1)
            drain_scatters(0)

            @pl.when(i < ngp - 1)
            def _():
                fire_gathers(j0 + 2 * G, 0)

            fire_scatters(j0 + G, 1)
            return carry

        lax.fori_loop(0, ngp, step, 0)
        drain_scatters(1)

    for zref, aggout in ((z_lo, agg_lo), (z_hi, agg_hi)):
        for k in range(4):
            pltpu.sync_copy(zbuf, agg_sh.at[pl.ds(base + k * CH, CH)])
        pltpu.sync_copy(zbuf.at[pl.ds(0, RPT - 4 * CH)],
                        agg_sh.at[pl.ds(base + 4 * CH, RPT - 4 * CH)])
        plsc.subcore_barrier()

        @pl.when(cid == 0)
        def _run_a():
            run_pipeline(zref, CPT_A)

        @pl.when(cid == 1)
        def _run_b():
            run_pipeline(zref, CPT_B)

        plsc.subcore_barrier()
        pltpu.sync_copy(agg_sh.at[pl.ds(base, RPT)],
                        aggout.at[cid, pl.ds(base, RPT)])
        plsc.subcore_barrier()


def _sc_agg(z_lo, z_hi, row3d, col3d, zeros64):
    mesh = plsc.VectorSubcoreMesh(core_axis_name="c", subcore_axis_name="s")
    return pl.kernel(
        _sc_agg_body,
        out_type=(
            jax.ShapeDtypeStruct((NC, AGG_ROWS, HD), F32),
            jax.ShapeDtypeStruct((NC, AGG_ROWS, HD), F32),
        ),
        mesh=mesh,
        compiler_params=pltpu.CompilerParams(use_tc_tiling_on_sc=False),
        scratch_types=(
            pltpu.VMEM_SHARED((AGG_ROWS, HD), F32),
            pltpu.VMEM((CPT_A, CH), jnp.int32),
            pltpu.VMEM((CPT_A, CH), jnp.int32),
            pltpu.VMEM((2 * G, CH, HD), F32),
            pltpu.VMEM((CH, HD), F32),
            pltpu.SemaphoreType.DMA,
            pltpu.SemaphoreType.DMA,
        ),
    )(z_lo, z_hi, row3d, col3d, zeros64)


def _tc_prep_body(dm_ref, x_ref, zlo_ref, zhi_ref, dinv_ref):
    dm = dm_ref[...]
    deg = dm[0, :, 0:1] + dm[1, :, 0:1]
    dinv = jnp.where(deg > 0.0, lax.rsqrt(jnp.maximum(deg, 1.0)), 0.0)
    z = x_ref[...] * dinv
    zlo_ref[...] = z[:, :HD]
    zhi_ref[...] = z[:, HD:]
    dinv_ref[...] = dinv


def _tc_prep(degmat, x_g1):
    blk = 1000
    grid = N1 // blk
    return pl.pallas_call(
        _tc_prep_body,
        grid=(grid,),
        in_specs=[
            pl.BlockSpec((NC, blk, 16), lambda i: (0, i, 0)),
            pl.BlockSpec((blk, D), lambda i: (i, 0)),
        ],
        out_specs=[
            pl.BlockSpec((blk, HD), lambda i: (i, 0)),
            pl.BlockSpec((blk, HD), lambda i: (i, 0)),
            pl.BlockSpec((blk, 1), lambda i: (i, 0)),
        ],
        out_shape=[
            jax.ShapeDtypeStruct((N1, HD), F32),
            jax.ShapeDtypeStruct((N1, HD), F32),
            jax.ShapeDtypeStruct((N1, 1), F32),
        ],
    )(degmat, x_g1)


def _tc_out1_body(alo_ref, ahi_ref, dinv_ref, w_ref, b_ref, o_ref):
    a = jnp.concatenate(
        [alo_ref[0] + alo_ref[1], ahi_ref[0] + ahi_ref[1]], axis=1)
    a = a * dinv_ref[...]
    h = jnp.dot(a, w_ref[...], preferred_element_type=F32,
                precision=_HIGH) + b_ref[...]
    h = jnp.maximum(h, 0.0)
    t = h - jnp.max(h, axis=1, keepdims=True)
    o_ref[...] = t - jnp.log(jnp.sum(jnp.exp(t), axis=1, keepdims=True))


def _tc_out1(agg_lo, agg_hi, dinv, W1, b1):
    blk = 1000
    grid = N1 // blk
    return pl.pallas_call(
        _tc_out1_body,
        grid=(grid,),
        in_specs=[
            pl.BlockSpec((NC, blk, HD), lambda i: (0, i, 0)),
            pl.BlockSpec((NC, blk, HD), lambda i: (0, i, 0)),
            pl.BlockSpec((blk, 1), lambda i: (i, 0)),
            pl.BlockSpec((D, D), lambda i: (0, 0)),
            pl.BlockSpec((1, D), lambda i: (0, 0)),
        ],
        out_specs=pl.BlockSpec((blk, D), lambda i: (i, 0)),
        out_shape=jax.ShapeDtypeStruct((N1, D), F32),
    )(agg_lo, agg_hi, dinv, W1, b1)


def _tc_xb_body(x_ref, b_ref, o_ref):
    gidx = pl.program_id(0)
    x = x_ref[...]
    parts = []
    for bb in range(8):
        r = jnp.dot(x, b_ref[bb], preferred_element_type=F32,
                    precision=_HIGH)
        r = jnp.where(gidx * 8 + bb < NB, r, 0.0)
        parts.append(r[:, None, :])
    o_ref[...] = jnp.concatenate(parts, axis=1)


def _tc_xb(x_g2, bases):
    return pl.pallas_call(
        _tc_xb_body,
        grid=(NBP // 8,),
        in_specs=[
            pl.BlockSpec((N2, N2), lambda i: (0, 0)),
            pl.BlockSpec((8, N2, N2), lambda i: (i, 0, 0)),
        ],
        out_specs=pl.BlockSpec((N2, 8, N2), lambda i: (0, i, 0)),
        out_shape=jax.ShapeDtypeStruct((N2, NBP, N2), F32),
    )(x_g2, bases)


def _tc_out2_body(a4_ref, xb3_ref, x_ref, root_ref, cnt_ref, b2_ref, o_ref):
    s = jnp.dot(a4_ref[...], xb3_ref[...], preferred_element_type=F32,
                precision=_HIGH)
    cnt = jnp.maximum(cnt_ref[...][:, 0:1], 1.0)
    h = s / cnt + jnp.dot(x_ref[...], root_ref[...],
                          preferred_element_type=F32,
                          precision=_HIGH) + b2_ref[...]
    h = jnp.maximum(h, 0.0)
    t = h - jnp.max(h, axis=1, keepdims=True)
    o_ref[...] = t - jnp.log(jnp.sum(jnp.exp(t), axis=1, keepdims=True))


def _tc_out2(A4, xb3, x_g2, root, cntmat, b2):
    return pl.pallas_call(
        _tc_out2_body,
        out_shape=jax.ShapeDtypeStruct((N2, N2), F32),
    )(A4, xb3, x_g2, root, cntmat, b2)


def kernel(x_g1, edge_index_g1, W1, b1, x_g2, edge_index_g2, edge_type_g2,
           bases, comp, root, b2):
    i32 = jnp.int32
    pad1 = E1P - E1
    row3d = jnp.concatenate(
        [edge_index_g1[0], jnp.zeros((pad1,), i32)]).reshape(NW * CPT1, CH)
    col3d = jnp.concatenate(
        [edge_index_g1[1],
         N1 + jnp.arange(pad1, dtype=i32) % (AGG_ROWS - N1)]
    ).reshape(NW * CPT1, CH)

    etype2d = edge_type_g2.reshape(NS, EPT2)
    src2d = edge_index_g2[0].reshape(NS, EPT2)
    dst2d = edge_index_g2[1].reshape(NS, EPT2)

    comp_pad = jnp.pad(comp, ((0, 0), (0, NBP - NB)))
    comp8 = jnp.concatenate(
        [comp_pad[:, q * 8:(q + 1) * 8] for q in range(4)], axis=0)

    zeros16 = jnp.zeros((RPT, 16), F32)
    zeros8 = jnp.zeros((A_RPT, 8), F32)
    lane = lax.broadcasted_iota(i32, (CH, 16), 1)
    ones16 = jnp.where(lane == 0, 1.0, 0.0).astype(F32)
    zeros64 = jnp.zeros((CH, HD), F32)

    degmat, a_raw, cntmat = _sc_count(
        col3d, etype2d, dst2d, src2d, comp8, zeros16, zeros8, ones16)

    z_lo, z_hi, dinv = _tc_prep(degmat, x_g1)

    agg_lo, agg_hi = _sc_agg(z_lo, z_hi, row3d, col3d, zeros64)

    out1 = _tc_out1(agg_lo, agg_hi, dinv, W1, b1.reshape(1, D))

    xb2 = _tc_xb(x_g2, bases)
    xb3 = xb2.reshape(NBP * N2, N2)
    A4 = a_raw.reshape(N2, N2 * NBP)
    out2 = _tc_out2(A4, xb3, x_g2, root, cntmat, b2.reshape(1, N2))

    return (out1, out2)


# speedup vs baseline: 19.3876x; 2.0548x over previous
import jax
import jax.numpy as jnp
from jax import lax
from jax.experimental import pallas as pl
from jax.experimental.pallas import tpu as pltpu
from jax.experimental.pallas import tpu_sc as plsc

F32 = jnp.float32

N1 = 10000
E1 = 320000
D = 128
HD = D // 2
N2 = 256
E2 = 2048
NUM_REL = 2048
NB = 30
NBP = 32

NC = 2
NS = 16
NW = NC * NS

CH = 128
CPT1 = 80
EPT1 = CPT1 * CH
E1P = EPT1 * NW

RPT = 632
AGG_ROWS = NS * RPT

A_ROWS = N2 * N2
A_RPT = A_ROWS // NS
EPT2 = E2 // NS

_HIGH = lax.Precision.HIGHEST


def _sc_count_body(col3d, etype2d, dst2d, src2d, comp8, zeros16, zeros8, ones16,
                   deg_out, a_out, cnt_out,
                   degmat_sh, a_sh, cnt_sh,
                   colbuf, compbuf, onesbuf, zbuf, zbuf8, ebuf, dbuf, sbuf,
                   fbuf, dsem):
    cid = lax.axis_index("c")
    sid = lax.axis_index("s")
    wid = cid * NS + sid

    pltpu.sync_copy(zeros16, zbuf)
    pltpu.sync_copy(zeros8, zbuf8)
    pltpu.sync_copy(ones16, onesbuf)
    pltpu.sync_copy(zbuf, degmat_sh.at[pl.ds(sid * RPT, RPT)])

    @pl.when(jnp.logical_and(cid == 0, sid == 0))
    def _zero_cnt():
        pltpu.sync_copy(zbuf.at[pl.ds(0, N2)], cnt_sh)

    pltpu.sync_copy(col3d.at[pl.ds(wid * CPT1, CPT1)], colbuf)
    pltpu.sync_copy(etype2d.at[sid], ebuf.at[0])
    pltpu.sync_copy(dst2d.at[sid], dbuf.at[0])
    pltpu.sync_copy(src2d.at[sid], sbuf.at[0])

    for k in range(EPT2 // 16):
        sl = pl.ds(k * 16, 16)
        fbuf[0, sl] = dbuf[0, sl] * N2 + sbuf[0, sl]

    plsc.subcore_barrier()

    def deg_step(i, carry):
        for k in range(8):
            pltpu.async_copy(onesbuf, degmat_sh.at[colbuf.at[i * 8 + k]],
                             dsem, add=True)
        for k in range(8):
            pltpu.make_async_copy(
                onesbuf, degmat_sh.at[colbuf.at[i * 8 + k]], dsem).wait()
        return carry

    lax.fori_loop(0, CPT1 // 8, deg_step, 0)

    @pl.when(cid == 0)
    def _cnt():
        pltpu.sync_copy(onesbuf, cnt_sh.at[dbuf.at[0]], add=True)

    for g in range(2):
        gidx = cid * 2 + g
        for k in range(EPT2 // 16):
            sl = pl.ds(k * 16, 16)
            ebuf[1, sl] = ebuf[0, sl] + gidx * NUM_REL
        pltpu.sync_copy(zbuf8, a_sh.at[pl.ds(sid * A_RPT, A_RPT)])
        plsc.subcore_barrier()
        pltpu.sync_copy(comp8.at[ebuf.at[1]], compbuf)
        pltpu.sync_copy(compbuf, a_sh.at[fbuf.at[0]], add=True)
        plsc.subcore_barrier()
        pltpu.sync_copy(a_sh.at[pl.ds(sid * A_RPT, A_RPT)],
                        a_out.at[pl.ds(sid * A_RPT, A_RPT),
                                 pl.ds(gidx * 8, 8)])

    plsc.subcore_barrier()
    pltpu.sync_copy(degmat_sh.at[pl.ds(sid * RPT, RPT)],
                    deg_out.at[cid, pl.ds(sid * RPT, RPT)])

    @pl.when(cid == 0)
    def _dump_cnt():
        pltpu.sync_copy(cnt_sh.at[pl.ds(sid * 16, 16)],
                        cnt_out.at[pl.ds(sid * 16, 16)])


def _sc_count(col3d, etype2d, dst2d, src2d, comp8, zeros16, zeros8, ones16):
    mesh = plsc.VectorSubcoreMesh(core_axis_name="c", subcore_axis_name="s")
    return pl.kernel(
        _sc_count_body,
        out_type=(
            jax.ShapeDtypeStruct((NC, AGG_ROWS, 16), F32),
            jax.ShapeDtypeStruct((A_ROWS, NBP), F32),
            jax.ShapeDtypeStruct((N2, 16), F32),
        ),
        mesh=mesh,
        compiler_params=pltpu.CompilerParams(use_tc_tiling_on_sc=False),
        scratch_types=(
            pltpu.VMEM_SHARED((AGG_ROWS, 16), F32),
            pltpu.VMEM_SHARED((A_ROWS, 8), F32),
            pltpu.VMEM_SHARED((N2, 16), F32),
            pltpu.VMEM((CPT1, CH), jnp.int32),
            pltpu.VMEM((CH, 8), F32),
            pltpu.VMEM((CH, 16), F32),
            pltpu.VMEM((RPT, 16), F32),
            pltpu.VMEM((A_RPT, 8), F32),
            pltpu.VMEM((2, CH), jnp.int32),
            pltpu.VMEM((1, CH), jnp.int32),
            pltpu.VMEM((1, CH), jnp.int32),
            pltpu.VMEM((1, CH), jnp.int32),
            pltpu.SemaphoreType.DMA,
        ),
    )(col3d, etype2d, dst2d, src2d, comp8, zeros16, zeros8, ones16)


G = 2
CPT_A = 80
CPT_B = 80


def _sc_agg_body(z_lo, z_hi, row2d, col2d, zeros64,
                 agg_lo, agg_hi,
                 agg_sh, rowbuf, colbuf, gbufs, zbuf, gsem, ssem):
    cid = lax.axis_index("c")
    sid = lax.axis_index("s")
    base = sid * RPT

    pltpu.sync_copy(zeros64, zbuf)

    @pl.when(cid == 0)
    def _stage_a():
        pltpu.sync_copy(row2d.at[pl.ds(sid * CPT_A, CPT_A)], rowbuf)
        pltpu.sync_copy(col2d.at[pl.ds(sid * CPT_A, CPT_A)], colbuf)

    @pl.when(cid == 1)
    def _stage_b():
        pltpu.sync_copy(row2d.at[pl.ds(NS * CPT_A + sid * CPT_B, CPT_B)],
                        rowbuf.at[pl.ds(0, CPT_B)])
        pltpu.sync_copy(col2d.at[pl.ds(NS * CPT_A + sid * CPT_B, CPT_B)],
                        colbuf.at[pl.ds(0, CPT_B)])

    def run_pipeline(zref, cpt):
        ngp = cpt // (2 * G)

        def fire_gathers(j0, bufset):
            for k in range(G):
                pltpu.async_copy(zref.at[rowbuf.at[j0 + k]],
                                 gbufs.at[bufset * G + k], gsem)

        def drain_gathers(bufset):
            for k in range(G):
                pltpu.make_async_copy(zref.at[rowbuf.at[0]],
                                      gbufs.at[bufset * G + k], gsem).wait()

        def fire_scatters(j0, bufset):
            for k in range(G):
                pltpu.async_copy(gbufs.at[bufset * G + k],
                                 agg_sh.at[colbuf.at[j0 + k]], ssem, add=True)

        def drain_scatters(bufset):
            for k in range(G):
                pltpu.make_async_copy(gbufs.at[bufset * G + k],
                                      agg_sh.at[colbuf.at[0]], ssem).wait()

        fire_gathers(0, 0)

        def step(i, carry):
            j0 = 2 * G * i
            drain_gathers(0)

            @pl.when(i > 0)
            def _():
                drain_scatters(1)

            fire_gathers(j0 + G, 1)
            fire_scatters(j0, 0)
            drain_gathers(1)
            drain_scatters(0)

            @pl.when(i < ngp - 1)
            def _():
                fire_gathers(j0 + 2 * G, 0)

            fire_scatters(j0 + G, 1)
            return carry

        lax.fori_loop(0, ngp, step, 0)
        drain_scatters(1)

    for zref, aggout in ((z_lo, agg_lo), (z_hi, agg_hi)):
        for k in range(4):
            pltpu.sync_copy(zbuf, agg_sh.at[pl.ds(base + k * CH, CH)])
        pltpu.sync_copy(zbuf.at[pl.ds(0, RPT - 4 * CH)],
                        agg_sh.at[pl.ds(base + 4 * CH, RPT - 4 * CH)])
        plsc.subcore_barrier()

        @pl.when(cid == 0)
        def _run_a():
            run_pipeline(zref, CPT_A)

        @pl.when(cid == 1)
        def _run_b():
            run_pipeline(zref, CPT_B)

        plsc.subcore_barrier()
        pltpu.sync_copy(agg_sh.at[pl.ds(base, RPT)],
                        aggout.at[cid, pl.ds(base, RPT)])
        plsc.subcore_barrier()


def _sc_agg(z_lo, z_hi, row3d, col3d, zeros64):
    mesh = plsc.VectorSubcoreMesh(core_axis_name="c", subcore_axis_name="s")
    return pl.kernel(
        _sc_agg_body,
        out_type=(
            jax.ShapeDtypeStruct((NC, AGG_ROWS, HD), F32),
            jax.ShapeDtypeStruct((NC, AGG_ROWS, HD), F32),
        ),
        mesh=mesh,
        compiler_params=pltpu.CompilerParams(use_tc_tiling_on_sc=False),
        scratch_types=(
            pltpu.VMEM_SHARED((AGG_ROWS, HD), F32),
            pltpu.VMEM((CPT_A, CH), jnp.int32),
            pltpu.VMEM((CPT_A, CH), jnp.int32),
            pltpu.VMEM((2 * G, CH, HD), F32),
            pltpu.VMEM((CH, HD), F32),
            pltpu.SemaphoreType.DMA,
            pltpu.SemaphoreType.DMA,
        ),
    )(z_lo, z_hi, row3d, col3d, zeros64)


def _tc_prep_body(dm_ref, x_ref, zlo_ref, zhi_ref, dinv_ref):
    dm = dm_ref[...]
    deg = dm[0, :, 0:1] + dm[1, :, 0:1]
    dinv = jnp.where(deg > 0.0, lax.rsqrt(jnp.maximum(deg, 1.0)), 0.0)
    z = x_ref[...] * dinv
    zlo_ref[...] = z[:, :HD]
    zhi_ref[...] = z[:, HD:]
    dinv_ref[...] = dinv


def _tc_prep(degmat, x_g1):
    blk = 1000
    grid = N1 // blk
    return pl.pallas_call(
        _tc_prep_body,
        grid=(grid,),
        in_specs=[
            pl.BlockSpec((NC, blk, 16), lambda i: (0, i, 0)),
            pl.BlockSpec((blk, D), lambda i: (i, 0)),
        ],
        out_specs=[
            pl.BlockSpec((blk, HD), lambda i: (i, 0)),
            pl.BlockSpec((blk, HD), lambda i: (i, 0)),
            pl.BlockSpec((blk, 1), lambda i: (i, 0)),
        ],
        out_shape=[
            jax.ShapeDtypeStruct((N1, HD), F32),
            jax.ShapeDtypeStruct((N1, HD), F32),
            jax.ShapeDtypeStruct((N1, 1), F32),
        ],
    )(degmat, x_g1)


def _tc_out1_body(alo_ref, ahi_ref, dinv_ref, w_ref, b_ref, o_ref):
    a = jnp.concatenate(
        [alo_ref[0] + alo_ref[1], ahi_ref[0] + ahi_ref[1]], axis=1)
    a = a * dinv_ref[...]
    h = jnp.dot(a, w_ref[...], preferred_element_type=F32,
                precision=_HIGH) + b_ref[...]
    h = jnp.maximum(h, 0.0)
    t = h - jnp.max(h, axis=1, keepdims=True)
    o_ref[...] = t - jnp.log(jnp.sum(jnp.exp(t), axis=1, keepdims=True))


def _tc_out1(agg_lo, agg_hi, dinv, W1, b1):
    blk = 1000
    grid = N1 // blk
    return pl.pallas_call(
        _tc_out1_body,
        grid=(grid,),
        in_specs=[
            pl.BlockSpec((NC, blk, HD), lambda i: (0, i, 0)),
            pl.BlockSpec((NC, blk, HD), lambda i: (0, i, 0)),
            pl.BlockSpec((blk, 1), lambda i: (i, 0)),
            pl.BlockSpec((D, D), lambda i: (0, 0)),
            pl.BlockSpec((1, D), lambda i: (0, 0)),
        ],
        out_specs=pl.BlockSpec((blk, D), lambda i: (i, 0)),
        out_shape=jax.ShapeDtypeStruct((N1, D), F32),
    )(agg_lo, agg_hi, dinv, W1, b1)


def _tc_xb_body(x_ref, b_ref, o_ref):
    gidx = pl.program_id(0)
    x = x_ref[...]
    parts = []
    for bb in range(8):
        r = jnp.dot(x, b_ref[bb], preferred_element_type=F32,
                    precision=_HIGH)
        r = jnp.where(gidx * 8 + bb < NB, r, 0.0)
        parts.append(r[:, None, :])
    o_ref[...] = jnp.concatenate(parts, axis=1)


def _tc_xb(x_g2, bases):
    return pl.pallas_call(
        _tc_xb_body,
        grid=(NBP // 8,),
        in_specs=[
            pl.BlockSpec((N2, N2), lambda i: (0, 0)),
            pl.BlockSpec((8, N2, N2), lambda i: (i, 0, 0)),
        ],
        out_specs=pl.BlockSpec((N2, 8, N2), lambda i: (0, i, 0)),
        out_shape=jax.ShapeDtypeStruct((N2, NBP, N2), F32),
    )(x_g2, bases)


def _tc_out2_body(a4_ref, xb3_ref, x_ref, root_ref, cnt_ref, b2_ref, o_ref):
    s = jnp.dot(a4_ref[...], xb3_ref[...], preferred_element_type=F32,
                precision=_HIGH)
    cnt = jnp.maximum(cnt_ref[...][:, 0:1], 1.0)
    h = s / cnt + jnp.dot(x_ref[...], root_ref[...],
                          preferred_element_type=F32,
                          precision=_HIGH) + b2_ref[...]
    h = jnp.maximum(h, 0.0)
    t = h - jnp.max(h, axis=1, keepdims=True)
    o_ref[...] = t - jnp.log(jnp.sum(jnp.exp(t), axis=1, keepdims=True))


def _tc_out2(A4, xb3, x_g2, root, cntmat, b2):
    return pl.pallas_call(
        _tc_out2_body,
        out_shape=jax.ShapeDtypeStruct((N2, N2), F32),
    )(A4, xb3, x_g2, root, cntmat, b2)


def kernel(x_g1, edge_index_g1, W1, b1, x_g2, edge_index_g2, edge_type_g2,
           bases, comp, root, b2):
    i32 = jnp.int32
    pad1 = E1P - E1
    row3d = jnp.concatenate(
        [edge_index_g1[0],
         jnp.arange(pad1, dtype=i32) * 37 % N1]).reshape(NW * CPT1, CH)
    col3d = jnp.concatenate(
        [edge_index_g1[1],
         N1 + jnp.arange(pad1, dtype=i32) % (AGG_ROWS - N1)]
    ).reshape(NW * CPT1, CH)

    etype2d = edge_type_g2.reshape(NS, EPT2)
    src2d = edge_index_g2[0].reshape(NS, EPT2)
    dst2d = edge_index_g2[1].reshape(NS, EPT2)

    comp_pad = jnp.pad(comp, ((0, 0), (0, NBP - NB)))
    comp8 = jnp.concatenate(
        [comp_pad[:, q * 8:(q + 1) * 8] for q in range(4)], axis=0)

    zeros16 = jnp.zeros((RPT, 16), F32)
    zeros8 = jnp.zeros((A_RPT, 8), F32)
    lane = lax.broadcasted_iota(i32, (CH, 16), 1)
    ones16 = jnp.where(lane == 0, 1.0, 0.0).astype(F32)
    zeros64 = jnp.zeros((CH, HD), F32)

    degmat, a_raw, cntmat = _sc_count(
        col3d, etype2d, dst2d, src2d, comp8, zeros16, zeros8, ones16)

    z_lo, z_hi, dinv = _tc_prep(degmat, x_g1)

    agg_lo, agg_hi = _sc_agg(z_lo, z_hi, row3d, col3d, zeros64)

    out1 = _tc_out1(agg_lo, agg_hi, dinv, W1, b1.reshape(1, D))

    xb2 = _tc_xb(x_g2, bases)
    xb3 = xb2.reshape(NBP * N2, N2)
    A4 = a_raw.reshape(N2, N2 * NBP)
    out2 = _tc_out2(A4, xb3, x_g2, root, cntmat, b2.reshape(1, N2))

    return (out1, out2)


# speedup vs baseline: 21.0208x; 1.0842x over previous
import jax
import jax.numpy as jnp
from jax import lax
from jax.experimental import pallas as pl
from jax.experimental.pallas import tpu as pltpu
from jax.experimental.pallas import tpu_sc as plsc

F32 = jnp.float32

N1 = 10000
E1 = 320000
D = 128
HD = D // 2
N2 = 256
E2 = 2048
NUM_REL = 2048
NB = 30
NBP = 32

NC = 2
NS = 16
NW = NC * NS

CH = 128
CPT1 = 81
EPT1 = CPT1 * CH
E1P = EPT1 * NW

RPT = 632
AGG_ROWS = NS * RPT

A_ROWS = N2 * N2
A_RPT = A_ROWS // NS
EPT2 = E2 // NS

_HIGH = lax.Precision.HIGHEST


def _sc_count_body(col3d, etype2d, dst2d, src2d, comp8, zeros8, ones8,
                   deg_out, a_out, cnt_out,
                   degmat_sh, a_sh, cnt_sh,
                   colbuf, compbuf, onesbuf, zbuf8, ebuf, dbuf, sbuf,
                   fbuf, dsem):
    cid = lax.axis_index("c")
    sid = lax.axis_index("s")
    wid = cid * NS + sid

    pltpu.sync_copy(zeros8, zbuf8)
    pltpu.sync_copy(ones8, onesbuf)
    pltpu.sync_copy(zbuf8.at[pl.ds(0, RPT)], degmat_sh.at[pl.ds(sid * RPT, RPT)])

    @pl.when(jnp.logical_and(cid == 0, sid == 0))
    def _zero_cnt():
        pltpu.sync_copy(zbuf8.at[pl.ds(0, N2)], cnt_sh)

    pltpu.sync_copy(col3d.at[pl.ds(wid * CPT1, CPT1)], colbuf)
    pltpu.sync_copy(etype2d.at[sid], ebuf.at[0])
    pltpu.sync_copy(dst2d.at[sid], dbuf.at[0])
    pltpu.sync_copy(src2d.at[sid], sbuf.at[0])

    for k in range(EPT2 // 16):
        sl = pl.ds(k * 16, 16)
        fbuf[0, sl] = dbuf[0, sl] * N2 + sbuf[0, sl]

    plsc.subcore_barrier()

    def deg_step(i, carry):
        for k in range(9):
            pltpu.async_copy(onesbuf, degmat_sh.at[colbuf.at[i * 9 + k]],
                             dsem, add=True)
        for k in range(9):
            pltpu.make_async_copy(
                onesbuf, degmat_sh.at[colbuf.at[i * 9 + k]], dsem).wait()
        return carry

    lax.fori_loop(0, CPT1 // 9, deg_step, 0)

    @pl.when(cid == 0)
    def _cnt():
        pltpu.sync_copy(onesbuf, cnt_sh.at[dbuf.at[0]], add=True)

    for g in range(2):
        gidx = cid * 2 + g
        for k in range(EPT2 // 16):
            sl = pl.ds(k * 16, 16)
            ebuf[1, sl] = ebuf[0, sl] + gidx * NUM_REL
        pltpu.sync_copy(zbuf8, a_sh.at[pl.ds(sid * A_RPT, A_RPT)])
        plsc.subcore_barrier()
        pltpu.sync_copy(comp8.at[ebuf.at[1]], compbuf)
        pltpu.sync_copy(compbuf, a_sh.at[fbuf.at[0]], add=True)
        plsc.subcore_barrier()
        pltpu.sync_copy(a_sh.at[pl.ds(sid * A_RPT, A_RPT)],
                        a_out.at[pl.ds(sid * A_RPT, A_RPT),
                                 pl.ds(gidx * 8, 8)])

    plsc.subcore_barrier()
    pltpu.sync_copy(degmat_sh.at[pl.ds(sid * RPT, RPT)],
                    deg_out.at[cid, pl.ds(sid * RPT, RPT)])

    @pl.when(cid == 0)
    def _dump_cnt():
        pltpu.sync_copy(cnt_sh.at[pl.ds(sid * 16, 16)],
                        cnt_out.at[pl.ds(sid * 16, 16)])


def _sc_count(col3d, etype2d, dst2d, src2d, comp8, zeros8, ones8):
    mesh = plsc.VectorSubcoreMesh(core_axis_name="c", subcore_axis_name="s")
    return pl.kernel(
        _sc_count_body,
        out_type=(
            jax.ShapeDtypeStruct((NC, AGG_ROWS, 8), F32),
            jax.ShapeDtypeStruct((A_ROWS, NBP), F32),
            jax.ShapeDtypeStruct((N2, 8), F32),
        ),
        mesh=mesh,
        compiler_params=pltpu.CompilerParams(use_tc_tiling_on_sc=False),
        scratch_types=(
            pltpu.VMEM_SHARED((AGG_ROWS, 8), F32),
            pltpu.VMEM_SHARED((A_ROWS, 8), F32),
            pltpu.VMEM_SHARED((N2, 8), F32),
            pltpu.VMEM((CPT1, CH), jnp.int32),
            pltpu.VMEM((CH, 8), F32),
            pltpu.VMEM((CH, 8), F32),
            pltpu.VMEM((A_RPT, 8), F32),
            pltpu.VMEM((2, CH), jnp.int32),
            pltpu.VMEM((1, CH), jnp.int32),
            pltpu.VMEM((1, CH), jnp.int32),
            pltpu.VMEM((1, CH), jnp.int32),
            pltpu.SemaphoreType.DMA,
        ),
    )(col3d, etype2d, dst2d, src2d, comp8, zeros8, ones8)


G = 3
CPT_A = 84
CPT_B = 78


def _sc_agg_body(z_lo, z_hi, row2d, col2d, zeros64,
                 agg_out,
                 agg_sh, rowbuf, colbuf, gbufs, zbuf, gsem, ssem):
    cid = lax.axis_index("c")
    sid = lax.axis_index("s")
    base = sid * RPT

    pltpu.sync_copy(zeros64, zbuf)

    @pl.when(cid == 0)
    def _stage_a():
        pltpu.sync_copy(row2d.at[pl.ds(sid * CPT_A, CPT_A)], rowbuf)
        pltpu.sync_copy(col2d.at[pl.ds(sid * CPT_A, CPT_A)], colbuf)

    @pl.when(cid == 1)
    def _stage_b():
        pltpu.sync_copy(row2d.at[pl.ds(NS * CPT_A + sid * CPT_B, CPT_B)],
                        rowbuf.at[pl.ds(0, CPT_B)])
        pltpu.sync_copy(col2d.at[pl.ds(NS * CPT_A + sid * CPT_B, CPT_B)],
                        colbuf.at[pl.ds(0, CPT_B)])

    def run_pipeline(zref, cpt):
        ngp = cpt // (2 * G)

        def fire_gathers(j0, bufset):
            for k in range(G):
                pltpu.async_copy(zref.at[rowbuf.at[j0 + k]],
                                 gbufs.at[bufset * G + k], gsem)

        def drain_gathers(bufset):
            for k in range(G):
                pltpu.make_async_copy(zref.at[rowbuf.at[0]],
                                      gbufs.at[bufset * G + k], gsem).wait()

        def fire_scatters(j0, bufset):
            for k in range(G):
                pltpu.async_copy(gbufs.at[bufset * G + k],
                                 agg_sh.at[colbuf.at[j0 + k]], ssem, add=True)

        def drain_scatters(bufset):
            for k in range(G):
                pltpu.make_async_copy(gbufs.at[bufset * G + k],
                                      agg_sh.at[colbuf.at[0]], ssem).wait()

        fire_gathers(0, 0)

        def step(i, carry):
            j0 = 2 * G * i
            drain_gathers(0)

            @pl.when(i > 0)
            def _():
                drain_scatters(1)

            fire_gathers(j0 + G, 1)
            fire_scatters(j0, 0)
            drain_gathers(1)
            drain_scatters(0)

            @pl.when(i < ngp - 1)
            def _():
                fire_gathers(j0 + 2 * G, 0)

            fire_scatters(j0 + G, 1)
            return carry

        lax.fori_loop(0, ngp, step, 0)
        drain_scatters(1)

    for h, zref in enumerate((z_lo, z_hi)):
        for k in range(4):
            pltpu.sync_copy(zbuf, agg_sh.at[pl.ds(base + k * CH, CH)])
        pltpu.sync_copy(zbuf.at[pl.ds(0, RPT - 4 * CH)],
                        agg_sh.at[pl.ds(base + 4 * CH, RPT - 4 * CH)])
        plsc.subcore_barrier()

        @pl.when(cid == 0)
        def _run_a():
            run_pipeline(zref, CPT_A)

        @pl.when(cid == 1)
        def _run_b():
            run_pipeline(zref, CPT_B)

        plsc.subcore_barrier()
        pltpu.sync_copy(agg_sh.at[pl.ds(base, RPT)],
                        agg_out.at[cid, pl.ds(base, RPT), pl.ds(h * HD, HD)])
        plsc.subcore_barrier()


def _sc_agg(z_lo, z_hi, row3d, col3d, zeros64):
    mesh = plsc.VectorSubcoreMesh(core_axis_name="c", subcore_axis_name="s")
    return pl.kernel(
        _sc_agg_body,
        out_type=jax.ShapeDtypeStruct((NC, AGG_ROWS, D), F32),
        mesh=mesh,
        compiler_params=pltpu.CompilerParams(use_tc_tiling_on_sc=False),
        scratch_types=(
            pltpu.VMEM_SHARED((AGG_ROWS, HD), F32),
            pltpu.VMEM((CPT_A, CH), jnp.int32),
            pltpu.VMEM((CPT_A, CH), jnp.int32),
            pltpu.VMEM((2 * G, CH, HD), F32),
            pltpu.VMEM((CH, HD), F32),
            pltpu.SemaphoreType.DMA,
            pltpu.SemaphoreType.DMA,
        ),
    )(z_lo, z_hi, row3d, col3d, zeros64)


def _tc_prep_body(dm_ref, x_ref, zlo_ref, zhi_ref, dinv_ref):
    dm = dm_ref[...]
    deg = dm[0, :, 0:1] + dm[1, :, 0:1]
    dinv = jnp.where(deg > 0.0, lax.rsqrt(jnp.maximum(deg, 1.0)), 0.0)
    z = x_ref[...] * dinv
    zlo_ref[...] = z[:, :HD]
    zhi_ref[...] = z[:, HD:]
    dinv_ref[...] = dinv


def _tc_prep(degmat, x_g1):
    blk = 1000
    grid = N1 // blk
    return pl.pallas_call(
        _tc_prep_body,
        grid=(grid,),
        in_specs=[
            pl.BlockSpec((NC, blk, 8), lambda i: (0, i, 0)),
            pl.BlockSpec((blk, D), lambda i: (i, 0)),
        ],
        out_specs=[
            pl.BlockSpec((blk, HD), lambda i: (i, 0)),
            pl.BlockSpec((blk, HD), lambda i: (i, 0)),
            pl.BlockSpec((blk, 1), lambda i: (i, 0)),
        ],
        out_shape=[
            jax.ShapeDtypeStruct((N1, HD), F32),
            jax.ShapeDtypeStruct((N1, HD), F32),
            jax.ShapeDtypeStruct((N1, 1), F32),
        ],
    )(degmat, x_g1)


def _tc_out1_body(agg_ref, dinv_ref, w_ref, b_ref, o_ref):
    a = (agg_ref[0] + agg_ref[1]) * dinv_ref[...]
    h = jnp.dot(a, w_ref[...], preferred_element_type=F32,
                precision=_HIGH) + b_ref[...]
    h = jnp.maximum(h, 0.0)
    t = h - jnp.max(h, axis=1, keepdims=True)
    o_ref[...] = t - jnp.log(jnp.sum(jnp.exp(t), axis=1, keepdims=True))


def _tc_out1(aggp, dinv, W1, b1):
    blk = 1000
    grid = N1 // blk
    return pl.pallas_call(
        _tc_out1_body,
        grid=(grid,),
        in_specs=[
            pl.BlockSpec((NC, blk, D), lambda i: (0, i, 0)),
            pl.BlockSpec((blk, 1), lambda i: (i, 0)),
            pl.BlockSpec((D, D), lambda i: (0, 0)),
            pl.BlockSpec((1, D), lambda i: (0, 0)),
        ],
        out_specs=pl.BlockSpec((blk, D), lambda i: (i, 0)),
        out_shape=jax.ShapeDtypeStruct((N1, D), F32),
    )(aggp, dinv, W1, b1)


def _tc_xb_body(x_ref, b_ref, o_ref):
    gidx = pl.program_id(0)
    x = x_ref[...]
    parts = []
    for bb in range(8):
        r = jnp.dot(x, b_ref[bb], preferred_element_type=F32,
                    precision=_HIGH)
        r = jnp.where(gidx * 8 + bb < NB, r, 0.0)
        parts.append(r[:, None, :])
    o_ref[...] = jnp.concatenate(parts, axis=1)


def _tc_xb(x_g2, bases):
    return pl.pallas_call(
        _tc_xb_body,
        grid=(NBP // 8,),
        in_specs=[
            pl.BlockSpec((N2, N2), lambda i: (0, 0)),
            pl.BlockSpec((8, N2, N2), lambda i: (i, 0, 0)),
        ],
        out_specs=pl.BlockSpec((N2, 8, N2), lambda i: (0, i, 0)),
        out_shape=jax.ShapeDtypeStruct((N2, NBP, N2), F32),
    )(x_g2, bases)


def _tc_out2_body(a4_ref, xb3_ref, x_ref, root_ref, cnt_ref, b2_ref, o_ref):
    s = jnp.dot(a4_ref[...], xb3_ref[...], preferred_element_type=F32,
                precision=_HIGH)
    cnt = jnp.maximum(cnt_ref[...][:, 0:1], 1.0)
    h = s / cnt + jnp.dot(x_ref[...], root_ref[...],
                          preferred_element_type=F32,
                          precision=_HIGH) + b2_ref[...]
    h = jnp.maximum(h, 0.0)
    t = h - jnp.max(h, axis=1, keepdims=True)
    o_ref[...] = t - jnp.log(jnp.sum(jnp.exp(t), axis=1, keepdims=True))


def _tc_out2(A4, xb3, x_g2, root, cntmat, b2):
    return pl.pallas_call(
        _tc_out2_body,
        out_shape=jax.ShapeDtypeStruct((N2, N2), F32),
    )(A4, xb3, x_g2, root, cntmat, b2)


def kernel(x_g1, edge_index_g1, W1, b1, x_g2, edge_index_g2, edge_type_g2,
           bases, comp, root, b2):
    i32 = jnp.int32
    pad1 = E1P - E1
    row3d = jnp.concatenate(
        [edge_index_g1[0],
         jnp.arange(pad1, dtype=i32) * 37 % N1]).reshape(NW * CPT1, CH)
    col3d = jnp.concatenate(
        [edge_index_g1[1],
         N1 + jnp.arange(pad1, dtype=i32) % (AGG_ROWS - N1)]
    ).reshape(NW * CPT1, CH)

    etype2d = edge_type_g2.reshape(NS, EPT2)
    src2d = edge_index_g2[0].reshape(NS, EPT2)
    dst2d = edge_index_g2[1].reshape(NS, EPT2)

    comp_pad = jnp.pad(comp, ((0, 0), (0, NBP - NB)))
    comp8 = jnp.concatenate(
        [comp_pad[:, q * 8:(q + 1) * 8] for q in range(4)], axis=0)

    zeros8 = jnp.zeros((A_RPT, 8), F32)
    lane = lax.broadcasted_iota(i32, (CH, 8), 1)
    ones8 = jnp.where(lane == 0, 1.0, 0.0).astype(F32)
    zeros64 = jnp.zeros((CH, HD), F32)

    degmat, a_raw, cntmat = _sc_count(
        col3d, etype2d, dst2d, src2d, comp8, zeros8, ones8)

    z_lo, z_hi, dinv = _tc_prep(degmat, x_g1)

    aggp = _sc_agg(z_lo, z_hi, row3d, col3d, zeros64)

    out1 = _tc_out1(aggp, dinv, W1, b1.reshape(1, D))

    xb2 = _tc_xb(x_g2, bases)
    xb3 = xb2.reshape(NBP * N2, N2)
    A4 = a_raw.reshape(N2, N2 * NBP)
    out2 = _tc_out2(A4, xb3, x_g2, root, cntmat, b2.reshape(1, N2))

    return (out1, out2)


# speedup vs baseline: 24.8841x; 1.1838x over previous
import jax
import jax.numpy as jnp
from jax import lax
from jax.experimental import pallas as pl
from jax.experimental.pallas import tpu as pltpu
from jax.experimental.pallas import tpu_sc as plsc

F32 = jnp.float32

N1 = 10000
E1 = 320000
D = 128
HD = D // 2
N2 = 256
E2 = 2048
NUM_REL = 2048
NB = 30
NBP = 32

NC = 2
NS = 16
NW = NC * NS

CH = 128
CPT1 = 81
EPT1 = CPT1 * CH
E1P = EPT1 * NW

RPT = 632
AGG_ROWS = NS * RPT

A_ROWS = N2 * N2
A_RPT = A_ROWS // NS
EPT2 = E2 // NS

_HIGH = lax.Precision.HIGHEST


def _sc_count_body(col3d, etype2d, dst2d, src2d, comp8, zeros8, ones8,
                   deg_out, a_out, cnt_out,
                   degmat_sh, a_sh, cnt_sh,
                   colbuf, compbuf, onesbuf, zbuf8, ebuf, dbuf, sbuf,
                   fbuf, dsem):
    cid = lax.axis_index("c")
    sid = lax.axis_index("s")
    wid = cid * NS + sid

    pltpu.sync_copy(zeros8, zbuf8)
    pltpu.sync_copy(ones8, onesbuf)
    pltpu.sync_copy(zbuf8.at[pl.ds(0, RPT)], degmat_sh.at[pl.ds(sid * RPT, RPT)])

    @pl.when(jnp.logical_and(cid == 0, sid == 0))
    def _zero_cnt():
        pltpu.sync_copy(zbuf8.at[pl.ds(0, N2)], cnt_sh)

    pltpu.sync_copy(col3d.at[pl.ds(wid * CPT1, CPT1)], colbuf)
    pltpu.sync_copy(etype2d.at[sid], ebuf.at[0])
    pltpu.sync_copy(dst2d.at[sid], dbuf.at[0])
    pltpu.sync_copy(src2d.at[sid], sbuf.at[0])

    for k in range(EPT2 // 16):
        sl = pl.ds(k * 16, 16)
        fbuf[0, sl] = dbuf[0, sl] * N2 + sbuf[0, sl]

    plsc.subcore_barrier()

    def deg_step(i, carry):
        for k in range(9):
            pltpu.async_copy(onesbuf, degmat_sh.at[colbuf.at[i * 9 + k]],
                             dsem, add=True)
        for k in range(9):
            pltpu.make_async_copy(
                onesbuf, degmat_sh.at[colbuf.at[i * 9 + k]], dsem).wait()
        return carry

    lax.fori_loop(0, CPT1 // 9, deg_step, 0)

    @pl.when(cid == 0)
    def _cnt():
        pltpu.sync_copy(onesbuf, cnt_sh.at[dbuf.at[0]], add=True)

    for g in range(2):
        gidx = cid * 2 + g
        for k in range(EPT2 // 16):
            sl = pl.ds(k * 16, 16)
            ebuf[1, sl] = ebuf[0, sl] + gidx * NUM_REL
        pltpu.sync_copy(zbuf8, a_sh.at[pl.ds(sid * A_RPT, A_RPT)])
        plsc.subcore_barrier()
        pltpu.sync_copy(comp8.at[ebuf.at[1]], compbuf)
        pltpu.sync_copy(compbuf, a_sh.at[fbuf.at[0]], add=True)
        plsc.subcore_barrier()
        pltpu.sync_copy(a_sh.at[pl.ds(sid * A_RPT, A_RPT)],
                        a_out.at[gidx, pl.ds(sid * A_RPT, A_RPT)])

    plsc.subcore_barrier()
    pltpu.sync_copy(degmat_sh.at[pl.ds(sid * RPT, RPT)],
                    deg_out.at[cid, pl.ds(sid * RPT, RPT)])

    @pl.when(cid == 0)
    def _dump_cnt():
        pltpu.sync_copy(cnt_sh.at[pl.ds(sid * 16, 16)],
                        cnt_out.at[pl.ds(sid * 16, 16)])


def _sc_count(col3d, etype2d, dst2d, src2d, comp8, zeros8, ones8):
    mesh = plsc.VectorSubcoreMesh(core_axis_name="c", subcore_axis_name="s")
    return pl.kernel(
        _sc_count_body,
        out_type=(
            jax.ShapeDtypeStruct((NC, AGG_ROWS, 8), F32),
            jax.ShapeDtypeStruct((4, A_ROWS, 8), F32),
            jax.ShapeDtypeStruct((N2, 8), F32),
        ),
        mesh=mesh,
        compiler_params=pltpu.CompilerParams(use_tc_tiling_on_sc=False),
        scratch_types=(
            pltpu.VMEM_SHARED((AGG_ROWS, 8), F32),
            pltpu.VMEM_SHARED((A_ROWS, 8), F32),
            pltpu.VMEM_SHARED((N2, 8), F32),
            pltpu.VMEM((CPT1, CH), jnp.int32),
            pltpu.VMEM((CH, 8), F32),
            pltpu.VMEM((CH, 8), F32),
            pltpu.VMEM((A_RPT, 8), F32),
            pltpu.VMEM((2, CH), jnp.int32),
            pltpu.VMEM((1, CH), jnp.int32),
            pltpu.VMEM((1, CH), jnp.int32),
            pltpu.VMEM((1, CH), jnp.int32),
            pltpu.SemaphoreType.DMA,
        ),
    )(col3d, etype2d, dst2d, src2d, comp8, zeros8, ones8)


G = 3
CPT_A = 84
CPT_B = 78


def _sc_agg_body(z_lo, z_hi, row2d, col2d, zeros64,
                 agg_out,
                 agg_sh, rowbuf, colbuf, gbufs, zbuf, gsem, ssem):
    cid = lax.axis_index("c")
    sid = lax.axis_index("s")
    base = sid * RPT

    pltpu.sync_copy(zeros64, zbuf)

    @pl.when(cid == 0)
    def _stage_a():
        pltpu.sync_copy(row2d.at[pl.ds(sid * CPT_A, CPT_A)], rowbuf)
        pltpu.sync_copy(col2d.at[pl.ds(sid * CPT_A, CPT_A)], colbuf)

    @pl.when(cid == 1)
    def _stage_b():
        pltpu.sync_copy(row2d.at[pl.ds(NS * CPT_A + sid * CPT_B, CPT_B)],
                        rowbuf.at[pl.ds(0, CPT_B)])
        pltpu.sync_copy(col2d.at[pl.ds(NS * CPT_A + sid * CPT_B, CPT_B)],
                        colbuf.at[pl.ds(0, CPT_B)])

    def run_pipeline(zref, cpt):
        ngp = cpt // (2 * G)

        def fire_gathers(j0, bufset):
            for k in range(G):
                pltpu.async_copy(zref.at[rowbuf.at[j0 + k]],
                                 gbufs.at[bufset * G + k], gsem)

        def drain_gathers(bufset):
            for k in range(G):
                pltpu.make_async_copy(zref.at[rowbuf.at[0]],
                                      gbufs.at[bufset * G + k], gsem).wait()

        def fire_scatters(j0, bufset):
            for k in range(G):
                pltpu.async_copy(gbufs.at[bufset * G + k],
                                 agg_sh.at[colbuf.at[j0 + k]], ssem, add=True)

        def drain_scatters(bufset):
            for k in range(G):
                pltpu.make_async_copy(gbufs.at[bufset * G + k],
                                      agg_sh.at[colbuf.at[0]], ssem).wait()

        fire_gathers(0, 0)

        def step(i, carry):
            j0 = 2 * G * i
            drain_gathers(0)

            @pl.when(i > 0)
            def _():
                drain_scatters(1)

            fire_gathers(j0 + G, 1)
            fire_scatters(j0, 0)
            drain_gathers(1)
            drain_scatters(0)

            @pl.when(i < ngp - 1)
            def _():
                fire_gathers(j0 + 2 * G, 0)

            fire_scatters(j0 + G, 1)
            return carry

        lax.fori_loop(0, ngp, step, 0)
        drain_scatters(1)

    for h, zref in enumerate((z_lo, z_hi)):
        for k in range(4):
            pltpu.sync_copy(zbuf, agg_sh.at[pl.ds(base + k * CH, CH)])
        pltpu.sync_copy(zbuf.at[pl.ds(0, RPT - 4 * CH)],
                        agg_sh.at[pl.ds(base + 4 * CH, RPT - 4 * CH)])
        plsc.subcore_barrier()

        @pl.when(cid == 0)
        def _run_a():
            run_pipeline(zref, CPT_A)

        @pl.when(cid == 1)
        def _run_b():
            run_pipeline(zref, CPT_B)

        plsc.subcore_barrier()
        pltpu.sync_copy(agg_sh.at[pl.ds(base, RPT)],
                        agg_out.at[cid, pl.ds(base, RPT), pl.ds(h * HD, HD)])
        plsc.subcore_barrier()


def _sc_agg(z_lo, z_hi, row3d, col3d, zeros64):
    mesh = plsc.VectorSubcoreMesh(core_axis_name="c", subcore_axis_name="s")
    return pl.kernel(
        _sc_agg_body,
        out_type=jax.ShapeDtypeStruct((NC, AGG_ROWS, D), F32),
        mesh=mesh,
        compiler_params=pltpu.CompilerParams(use_tc_tiling_on_sc=False),
        scratch_types=(
            pltpu.VMEM_SHARED((AGG_ROWS, HD), F32),
            pltpu.VMEM((CPT_A, CH), jnp.int32),
            pltpu.VMEM((CPT_A, CH), jnp.int32),
            pltpu.VMEM((2 * G, CH, HD), F32),
            pltpu.VMEM((CH, HD), F32),
            pltpu.SemaphoreType.DMA,
            pltpu.SemaphoreType.DMA,
        ),
    )(z_lo, z_hi, row3d, col3d, zeros64)


def _tc_prep_body(dm_ref, x_ref, zlo_ref, zhi_ref, dinv_ref):
    dm = dm_ref[...]
    deg = dm[0, :, 0:1] + dm[1, :, 0:1]
    dinv = jnp.where(deg > 0.0, lax.rsqrt(jnp.maximum(deg, 1.0)), 0.0)
    z = x_ref[...] * dinv
    zlo_ref[...] = z[:, :HD]
    zhi_ref[...] = z[:, HD:]
    dinv_ref[...] = dinv


def _tc_prep(degmat, x_g1):
    blk = 1000
    grid = N1 // blk
    return pl.pallas_call(
        _tc_prep_body,
        grid=(grid,),
        in_specs=[
            pl.BlockSpec((NC, blk, 8), lambda i: (0, i, 0)),
            pl.BlockSpec((blk, D), lambda i: (i, 0)),
        ],
        out_specs=[
            pl.BlockSpec((blk, HD), lambda i: (i, 0)),
            pl.BlockSpec((blk, HD), lambda i: (i, 0)),
            pl.BlockSpec((blk, 1), lambda i: (i, 0)),
        ],
        out_shape=[
            jax.ShapeDtypeStruct((N1, HD), F32),
            jax.ShapeDtypeStruct((N1, HD), F32),
            jax.ShapeDtypeStruct((N1, 1), F32),
        ],
    )(degmat, x_g1)


def _tc_out1_body(agg_ref, dinv_ref, w_ref, b_ref, o_ref):
    a = (agg_ref[0] + agg_ref[1]) * dinv_ref[...]
    h = jnp.dot(a, w_ref[...], preferred_element_type=F32,
                precision=_HIGH) + b_ref[...]
    h = jnp.maximum(h, 0.0)
    t = h - jnp.max(h, axis=1, keepdims=True)
    o_ref[...] = t - jnp.log(jnp.sum(jnp.exp(t), axis=1, keepdims=True))


def _tc_out1(aggp, dinv, W1, b1):
    blk = 1000
    grid = N1 // blk
    return pl.pallas_call(
        _tc_out1_body,
        grid=(grid,),
        in_specs=[
            pl.BlockSpec((NC, blk, D), lambda i: (0, i, 0)),
            pl.BlockSpec((blk, 1), lambda i: (i, 0)),
            pl.BlockSpec((D, D), lambda i: (0, 0)),
            pl.BlockSpec((1, D), lambda i: (0, 0)),
        ],
        out_specs=pl.BlockSpec((blk, D), lambda i: (i, 0)),
        out_shape=jax.ShapeDtypeStruct((N1, D), F32),
    )(aggp, dinv, W1, b1)


def _tc_xb_body(x_ref, b_ref, o_ref):
    gidx = pl.program_id(0)
    x = x_ref[...]
    parts = []
    for bb in range(8):
        r = jnp.dot(x, b_ref[bb], preferred_element_type=F32,
                    precision=_HIGH)
        r = jnp.where(gidx * 8 + bb < NB, r, 0.0)
        parts.append(r[:, None, :])
    o_ref[...] = jnp.concatenate(parts, axis=1)[None]


def _tc_xb(x_g2, bases):
    return pl.pallas_call(
        _tc_xb_body,
        grid=(NBP // 8,),
        in_specs=[
            pl.BlockSpec((N2, N2), lambda i: (0, 0)),
            pl.BlockSpec((8, N2, N2), lambda i: (i, 0, 0)),
        ],
        out_specs=pl.BlockSpec((1, N2, 8, N2), lambda i: (i, 0, 0, 0)),
        out_shape=jax.ShapeDtypeStruct((4, N2, 8, N2), F32),
    )(x_g2, bases)


def _tc_out2_body(a4_ref, xb3_ref, x_ref, root_ref, cnt_ref, b2_ref, o_ref,
                  acc_ref):
    g = pl.program_id(0)
    part = jnp.dot(a4_ref[0], xb3_ref[0],
                   preferred_element_type=F32, precision=_HIGH)

    @pl.when(g == 0)
    def _():
        acc_ref[...] = part

    @pl.when(g > 0)
    def _():
        acc_ref[...] = acc_ref[...] + part

    @pl.when(g == 3)
    def _():
        s = acc_ref[...]
        cnt = jnp.maximum(cnt_ref[...][:, 0:1], 1.0)
        h = s / cnt + jnp.dot(x_ref[...], root_ref[...],
                              preferred_element_type=F32,
                              precision=_HIGH) + b2_ref[...]
        h = jnp.maximum(h, 0.0)
        t = h - jnp.max(h, axis=1, keepdims=True)
        o_ref[...] = t - jnp.log(jnp.sum(jnp.exp(t), axis=1, keepdims=True))


def _tc_out2(a_raw, xb2, x_g2, root, cntmat, b2):
    return pl.pallas_call(
        _tc_out2_body,
        grid=(4,),
        in_specs=[
            pl.BlockSpec((1, N2, N2 * 8), lambda g: (g, 0, 0)),
            pl.BlockSpec((1, 8 * N2, N2), lambda g: (g, 0, 0)),
            pl.BlockSpec((N2, N2), lambda g: (0, 0)),
            pl.BlockSpec((N2, N2), lambda g: (0, 0)),
            pl.BlockSpec((N2, 8), lambda g: (0, 0)),
            pl.BlockSpec((1, N2), lambda g: (0, 0)),
        ],
        out_specs=pl.BlockSpec((N2, N2), lambda g: (0, 0)),
        out_shape=jax.ShapeDtypeStruct((N2, N2), F32),
        scratch_shapes=[pltpu.VMEM((N2, N2), F32)],
    )(a_raw, xb2, x_g2, root, cntmat, b2)


def kernel(x_g1, edge_index_g1, W1, b1, x_g2, edge_index_g2, edge_type_g2,
           bases, comp, root, b2):
    i32 = jnp.int32
    pad1 = E1P - E1
    row3d = jnp.concatenate(
        [edge_index_g1[0],
         jnp.arange(pad1, dtype=i32) * 37 % N1]).reshape(NW * CPT1, CH)
    col3d = jnp.concatenate(
        [edge_index_g1[1],
         N1 + jnp.arange(pad1, dtype=i32) % (AGG_ROWS - N1)]
    ).reshape(NW * CPT1, CH)

    etype2d = edge_type_g2.reshape(NS, EPT2)
    src2d = edge_index_g2[0].reshape(NS, EPT2)
    dst2d = edge_index_g2[1].reshape(NS, EPT2)

    comp_pad = jnp.pad(comp, ((0, 0), (0, NBP - NB)))
    comp8 = jnp.concatenate(
        [comp_pad[:, q * 8:(q + 1) * 8] for q in range(4)], axis=0)

    zeros8 = jnp.zeros((A_RPT, 8), F32)
    lane = lax.broadcasted_iota(i32, (CH, 8), 1)
    ones8 = jnp.where(lane == 0, 1.0, 0.0).astype(F32)
    zeros64 = jnp.zeros((CH, HD), F32)

    degmat, a_raw, cntmat = _sc_count(
        col3d, etype2d, dst2d, src2d, comp8, zeros8, ones8)

    z_lo, z_hi, dinv = _tc_prep(degmat, x_g1)

    aggp = _sc_agg(z_lo, z_hi, row3d, col3d, zeros64)

    out1 = _tc_out1(aggp, dinv, W1, b1.reshape(1, D))

    xb2 = _tc_xb(x_g2, bases).reshape(4, 8 * N2, N2)
    a_r = a_raw.reshape(4, N2, N2 * 8)
    out2 = _tc_out2(a_r, xb2, x_g2, root, cntmat, b2.reshape(1, N2))

    return (out1, out2)


# speedup vs baseline: 26.6539x; 1.0711x over previous
import jax
import jax.numpy as jnp
from jax import lax
from jax.experimental import pallas as pl
from jax.experimental.pallas import tpu as pltpu
from jax.experimental.pallas import tpu_sc as plsc

F32 = jnp.float32

N1 = 10000
E1 = 320000
D = 128
HD = D // 2
N2 = 256
E2 = 2048
NUM_REL = 2048
NB = 30
NBP = 32

NC = 2
NS = 16
NW = NC * NS

CH = 128
NCH = E1 // CH
CPT = 78
NXT = NCH - NW * CPT

RPT = 632
AGG_ROWS = NS * RPT

A_ROWS = N2 * N2
A_RPT = A_ROWS // NS
EPT2 = E2 // NS

_HIGH = lax.Precision.HIGHEST


def _sc_count_body(col3d, etype2d, dst2d, src2d, comp8, zeros8, ones8,
                   deg_out, a_out, cnt_out,
                   degmat_sh, a_sh, cnt_sh,
                   colbuf, compbuf, onesbuf, zbuf8, ebuf, dbuf, sbuf,
                   fbuf, dsem):
    cid = lax.axis_index("c")
    sid = lax.axis_index("s")
    wid = cid * NS + sid

    pltpu.sync_copy(zeros8, zbuf8)
    pltpu.sync_copy(ones8, onesbuf)
    pltpu.sync_copy(zbuf8.at[pl.ds(0, RPT)], degmat_sh.at[pl.ds(sid * RPT, RPT)])

    @pl.when(jnp.logical_and(cid == 0, sid == 0))
    def _zero_cnt():
        pltpu.sync_copy(zbuf8.at[pl.ds(0, N2)], cnt_sh)

    @pl.when(cid == 0)
    def _stage_c0():
        pltpu.sync_copy(col3d.at[pl.ds(sid * CPT, CPT)],
                        colbuf.at[pl.ds(0, CPT)])

    @pl.when(cid == 1)
    def _stage_c1():
        pltpu.sync_copy(col3d.at[pl.ds(NS * CPT + sid * CPT, CPT)],
                        colbuf.at[pl.ds(0, CPT)])

    @pl.when(jnp.logical_and(cid == 1, sid >= NS - NXT))
    def _stage_cx():
        pltpu.sync_copy(col3d.at[pl.ds(2 * NS * CPT + sid - (NS - NXT), 1)],
                        colbuf.at[pl.ds(CPT, 1)])
    pltpu.sync_copy(etype2d.at[sid], ebuf.at[0])
    pltpu.sync_copy(dst2d.at[sid], dbuf.at[0])
    pltpu.sync_copy(src2d.at[sid], sbuf.at[0])

    for k in range(EPT2 // 16):
        sl = pl.ds(k * 16, 16)
        fbuf[0, sl] = dbuf[0, sl] * N2 + sbuf[0, sl]

    plsc.subcore_barrier()

    def deg_step(i, carry):
        for k in range(6):
            pltpu.async_copy(onesbuf, degmat_sh.at[colbuf.at[i * 6 + k]],
                             dsem, add=True)
        for k in range(6):
            pltpu.make_async_copy(
                onesbuf, degmat_sh.at[colbuf.at[i * 6 + k]], dsem).wait()
        return carry

    lax.fori_loop(0, CPT // 6, deg_step, 0)

    @pl.when(jnp.logical_and(cid == 1, sid >= NS - NXT))
    def _deg_extra():
        pltpu.sync_copy(onesbuf, degmat_sh.at[colbuf.at[CPT]], add=True)

    @pl.when(cid == 0)
    def _cnt():
        pltpu.sync_copy(onesbuf, cnt_sh.at[dbuf.at[0]], add=True)

    for g in range(2):
        gidx = cid * 2 + g
        for k in range(EPT2 // 16):
            sl = pl.ds(k * 16, 16)
            ebuf[1, sl] = ebuf[0, sl] + gidx * NUM_REL
        pltpu.sync_copy(zbuf8, a_sh.at[pl.ds(sid * A_RPT, A_RPT)])
        plsc.subcore_barrier()
        pltpu.sync_copy(comp8.at[ebuf.at[1]], compbuf)
        pltpu.sync_copy(compbuf, a_sh.at[fbuf.at[0]], add=True)
        plsc.subcore_barrier()
        pltpu.sync_copy(a_sh.at[pl.ds(sid * A_RPT, A_RPT)],
                        a_out.at[gidx, pl.ds(sid * A_RPT, A_RPT)])

    plsc.subcore_barrier()
    pltpu.sync_copy(degmat_sh.at[pl.ds(sid * RPT, RPT)],
                    deg_out.at[cid, pl.ds(sid * RPT, RPT)])

    @pl.when(cid == 0)
    def _dump_cnt():
        pltpu.sync_copy(cnt_sh.at[pl.ds(sid * 16, 16)],
                        cnt_out.at[pl.ds(sid * 16, 16)])


def _sc_count(col3d, etype2d, dst2d, src2d, comp8, zeros8, ones8):
    mesh = plsc.VectorSubcoreMesh(core_axis_name="c", subcore_axis_name="s")
    return pl.kernel(
        _sc_count_body,
        out_type=(
            jax.ShapeDtypeStruct((NC, AGG_ROWS, 8), F32),
            jax.ShapeDtypeStruct((4, A_ROWS, 8), F32),
            jax.ShapeDtypeStruct((N2, 8), F32),
        ),
        mesh=mesh,
        compiler_params=pltpu.CompilerParams(use_tc_tiling_on_sc=False),
        scratch_types=(
            pltpu.VMEM_SHARED((AGG_ROWS, 8), F32),
            pltpu.VMEM_SHARED((A_ROWS, 8), F32),
            pltpu.VMEM_SHARED((N2, 8), F32),
            pltpu.VMEM((CPT + 1, CH), jnp.int32),
            pltpu.VMEM((CH, 8), F32),
            pltpu.VMEM((CH, 8), F32),
            pltpu.VMEM((A_RPT, 8), F32),
            pltpu.VMEM((2, CH), jnp.int32),
            pltpu.VMEM((1, CH), jnp.int32),
            pltpu.VMEM((1, CH), jnp.int32),
            pltpu.VMEM((1, CH), jnp.int32),
            pltpu.SemaphoreType.DMA,
        ),
    )(col3d, etype2d, dst2d, src2d, comp8, zeros8, ones8)


G = 3


def _sc_agg_body(z2, rows2, col2d, zeros64,
                 agg_out,
                 agg_sh, rowbuf, colbuf, gbufs, zbuf, gsem, ssem):
    cid = lax.axis_index("c")
    sid = lax.axis_index("s")
    base = sid * RPT

    pltpu.sync_copy(zeros64, zbuf)

    def stage_rows(h):
        hb = h * NCH

        @pl.when(cid == 0)
        def _stage_a():
            pltpu.sync_copy(rows2.at[pl.ds(hb + sid * CPT, CPT)],
                            rowbuf.at[pl.ds(0, CPT)])

        @pl.when(cid == 1)
        def _stage_b():
            pltpu.sync_copy(rows2.at[pl.ds(hb + NS * CPT + sid * CPT, CPT)],
                            rowbuf.at[pl.ds(0, CPT)])

        @pl.when(jnp.logical_and(cid == 1, sid >= NS - NXT))
        def _stage_x():
            pltpu.sync_copy(
                rows2.at[pl.ds(hb + 2 * NS * CPT + sid - (NS - NXT), 1)],
                rowbuf.at[pl.ds(CPT, 1)])

    @pl.when(cid == 0)
    def _stage_ca():
        pltpu.sync_copy(col2d.at[pl.ds(sid * CPT, CPT)],
                        colbuf.at[pl.ds(0, CPT)])

    @pl.when(cid == 1)
    def _stage_cb():
        pltpu.sync_copy(col2d.at[pl.ds(NS * CPT + sid * CPT, CPT)],
                        colbuf.at[pl.ds(0, CPT)])

    @pl.when(jnp.logical_and(cid == 1, sid >= NS - NXT))
    def _stage_cx2():
        pltpu.sync_copy(col2d.at[pl.ds(2 * NS * CPT + sid - (NS - NXT), 1)],
                        colbuf.at[pl.ds(CPT, 1)])

    def run_pipeline(zref, cpt):
        ngp = cpt // (2 * G)

        def fire_gathers(j0, bufset):
            for k in range(G):
                pltpu.async_copy(zref.at[rowbuf.at[j0 + k]],
                                 gbufs.at[bufset * G + k], gsem)

        def drain_gathers(bufset):
            for k in range(G):
                pltpu.make_async_copy(zref.at[rowbuf.at[0]],
                                      gbufs.at[bufset * G + k], gsem).wait()

        def fire_scatters(j0, bufset):
            for k in range(G):
                pltpu.async_copy(gbufs.at[bufset * G + k],
                                 agg_sh.at[colbuf.at[j0 + k]], ssem, add=True)

        def drain_scatters(bufset):
            for k in range(G):
                pltpu.make_async_copy(gbufs.at[bufset * G + k],
                                      agg_sh.at[colbuf.at[0]], ssem).wait()

        fire_gathers(0, 0)

        def step(i, carry):
            j0 = 2 * G * i
            drain_gathers(0)

            @pl.when(i > 0)
            def _():
                drain_scatters(1)

            fire_gathers(j0 + G, 1)
            fire_scatters(j0, 0)
            drain_gathers(1)
            drain_scatters(0)

            @pl.when(i < ngp - 1)
            def _():
                fire_gathers(j0 + 2 * G, 0)

            fire_scatters(j0 + G, 1)
            return carry

        lax.fori_loop(0, ngp, step, 0)
        drain_scatters(1)

    for h in range(2):
        stage_rows(h)
        for k in range(4):
            pltpu.sync_copy(zbuf, agg_sh.at[pl.ds(base + k * CH, CH)])
        pltpu.sync_copy(zbuf.at[pl.ds(0, RPT - 4 * CH)],
                        agg_sh.at[pl.ds(base + 4 * CH, RPT - 4 * CH)])
        plsc.subcore_barrier()

        run_pipeline(z2, CPT)

        @pl.when(jnp.logical_and(cid == 1, sid >= NS - NXT))
        def _extra():
            pltpu.sync_copy(z2.at[rowbuf.at[CPT]], gbufs.at[0])
            pltpu.sync_copy(gbufs.at[0], agg_sh.at[colbuf.at[CPT]], add=True)

        plsc.subcore_barrier()
        pltpu.sync_copy(agg_sh.at[pl.ds(base, RPT)],
                        agg_out.at[cid, pl.ds(base, RPT), pl.ds(h * HD, HD)])
        plsc.subcore_barrier()


def _sc_agg(z2, rows2, col2d, zeros64):
    mesh = plsc.VectorSubcoreMesh(core_axis_name="c", subcore_axis_name="s")
    return pl.kernel(
        _sc_agg_body,
        out_type=jax.ShapeDtypeStruct((NC, AGG_ROWS, D), F32),
        mesh=mesh,
        compiler_params=pltpu.CompilerParams(use_tc_tiling_on_sc=False),
        scratch_types=(
            pltpu.VMEM_SHARED((AGG_ROWS, HD), F32),
            pltpu.VMEM((CPT + 1, CH), jnp.int32),
            pltpu.VMEM((CPT + 1, CH), jnp.int32),
            pltpu.VMEM((2 * G, CH, HD), F32),
            pltpu.VMEM((CH, HD), F32),
            pltpu.SemaphoreType.DMA,
            pltpu.SemaphoreType.DMA,
        ),
    )(z2, rows2, col2d, zeros64)


def _tc_prep_body(dm_ref, x_ref, z_ref, dinv_ref):
    dm = dm_ref[...]
    deg = dm[0, :, 0:1] + dm[1, :, 0:1]
    dinv = jnp.where(deg > 0.0, lax.rsqrt(jnp.maximum(deg, 1.0)), 0.0)
    z_ref[...] = x_ref[...] * dinv
    dinv_ref[...] = dinv


def _tc_prep(degmat, x_g1):
    blk = 1000
    grid = N1 // blk
    return pl.pallas_call(
        _tc_prep_body,
        grid=(grid,),
        in_specs=[
            pl.BlockSpec((NC, blk, 8), lambda i: (0, i, 0)),
            pl.BlockSpec((blk, D), lambda i: (i, 0)),
        ],
        out_specs=[
            pl.BlockSpec((blk, D), lambda i: (i, 0)),
            pl.BlockSpec((blk, 1), lambda i: (i, 0)),
        ],
        out_shape=[
            jax.ShapeDtypeStruct((N1, D), F32),
            jax.ShapeDtypeStruct((N1, 1), F32),
        ],
    )(degmat, x_g1)


def _tc_out1_body(agg_ref, dinv_ref, w_ref, b_ref, o_ref):
    a = (agg_ref[0] + agg_ref[1]) * dinv_ref[...]
    h = jnp.dot(a, w_ref[...], preferred_element_type=F32,
                precision=_HIGH) + b_ref[...]
    h = jnp.maximum(h, 0.0)
    t = h - jnp.max(h, axis=1, keepdims=True)
    o_ref[...] = t - jnp.log(jnp.sum(jnp.exp(t), axis=1, keepdims=True))


def _tc_out1(aggp, dinv, W1, b1):
    blk = 1000
    grid = N1 // blk
    return pl.pallas_call(
        _tc_out1_body,
        grid=(grid,),
        in_specs=[
            pl.BlockSpec((NC, blk, D), lambda i: (0, i, 0)),
            pl.BlockSpec((blk, 1), lambda i: (i, 0)),
            pl.BlockSpec((D, D), lambda i: (0, 0)),
            pl.BlockSpec((1, D), lambda i: (0, 0)),
        ],
        out_specs=pl.BlockSpec((blk, D), lambda i: (i, 0)),
        out_shape=jax.ShapeDtypeStruct((N1, D), F32),
    )(aggp, dinv, W1, b1)


def _tc_xb_body(x_ref, b_ref, o_ref):
    gidx = pl.program_id(0)
    x = x_ref[...]
    parts = []
    for bb in range(8):
        r = jnp.dot(x, b_ref[bb], preferred_element_type=F32,
                    precision=_HIGH)
        r = jnp.where(gidx * 8 + bb < NB, r, 0.0)
        parts.append(r[:, None, :])
    o_ref[...] = jnp.concatenate(parts, axis=1)[None]


def _tc_xb(x_g2, bases):
    return pl.pallas_call(
        _tc_xb_body,
        grid=(NBP // 8,),
        in_specs=[
            pl.BlockSpec((N2, N2), lambda i: (0, 0)),
            pl.BlockSpec((8, N2, N2), lambda i: (i, 0, 0)),
        ],
        out_specs=pl.BlockSpec((1, N2, 8, N2), lambda i: (i, 0, 0, 0)),
        out_shape=jax.ShapeDtypeStruct((4, N2, 8, N2), F32),
    )(x_g2, bases)


def _tc_out2_body(a4_ref, xb3_ref, x_ref, root_ref, cnt_ref, b2_ref, o_ref,
                  acc_ref):
    g = pl.program_id(0)
    part = jnp.dot(a4_ref[0], xb3_ref[0],
                   preferred_element_type=F32, precision=_HIGH)

    @pl.when(g == 0)
    def _():
        acc_ref[...] = part

    @pl.when(g > 0)
    def _():
        acc_ref[...] = acc_ref[...] + part

    @pl.when(g == 3)
    def _():
        s = acc_ref[...]
        cnt = jnp.maximum(cnt_ref[...][:, 0:1], 1.0)
        h = s / cnt + jnp.dot(x_ref[...], root_ref[...],
                              preferred_element_type=F32,
                              precision=_HIGH) + b2_ref[...]
        h = jnp.maximum(h, 0.0)
        t = h - jnp.max(h, axis=1, keepdims=True)
        o_ref[...] = t - jnp.log(jnp.sum(jnp.exp(t), axis=1, keepdims=True))


def _tc_out2(a_raw, xb2, x_g2, root, cntmat, b2):
    return pl.pallas_call(
        _tc_out2_body,
        grid=(4,),
        in_specs=[
            pl.BlockSpec((1, N2, N2 * 8), lambda g: (g, 0, 0)),
            pl.BlockSpec((1, 8 * N2, N2), lambda g: (g, 0, 0)),
            pl.BlockSpec((N2, N2), lambda g: (0, 0)),
            pl.BlockSpec((N2, N2), lambda g: (0, 0)),
            pl.BlockSpec((N2, 8), lambda g: (0, 0)),
            pl.BlockSpec((1, N2), lambda g: (0, 0)),
        ],
        out_specs=pl.BlockSpec((N2, N2), lambda g: (0, 0)),
        out_shape=jax.ShapeDtypeStruct((N2, N2), F32),
        scratch_shapes=[pltpu.VMEM((N2, N2), F32)],
    )(a_raw, xb2, x_g2, root, cntmat, b2)


def kernel(x_g1, edge_index_g1, W1, b1, x_g2, edge_index_g2, edge_type_g2,
           bases, comp, root, b2):
    i32 = jnp.int32
    rlo = edge_index_g1[0] * 2
    rows2 = jnp.stack([rlo, rlo + 1]).reshape(2 * NCH, CH)
    col2d = edge_index_g1[1].reshape(NCH, CH)

    etype2d = edge_type_g2.reshape(NS, EPT2)
    src2d = edge_index_g2[0].reshape(NS, EPT2)
    dst2d = edge_index_g2[1].reshape(NS, EPT2)

    comp_pad = jnp.pad(comp, ((0, 0), (0, NBP - NB)))
    comp8 = jnp.concatenate(
        [comp_pad[:, q * 8:(q + 1) * 8] for q in range(4)], axis=0)

    zeros8 = jnp.zeros((A_RPT, 8), F32)
    lane = lax.broadcasted_iota(i32, (CH, 8), 1)
    ones8 = jnp.where(lane == 0, 1.0, 0.0).astype(F32)
    zeros64 = jnp.zeros((CH, HD), F32)

    degmat, a_raw, cntmat = _sc_count(
        col2d, etype2d, dst2d, src2d, comp8, zeros8, ones8)

    z, dinv = _tc_prep(degmat, x_g1)

    aggp = _sc_agg(z.reshape(2 * N1, HD), rows2, col2d, zeros64)

    out1 = _tc_out1(aggp, dinv, W1, b1.reshape(1, D))

    xb2 = _tc_xb(x_g2, bases).reshape(4, 8 * N2, N2)
    a_r = a_raw.reshape(4, N2, N2 * 8)
    out2 = _tc_out2(a_r, xb2, x_g2, root, cntmat, b2.reshape(1, N2))

    return (out1, out2)
